# baseline (speedup 1.0000x reference)
import functools

import jax
import jax.numpy as jnp
from jax import lax
from jax.experimental import pallas as pl
from jax.experimental.pallas import tpu as pltpu
from jax.experimental.pallas import tpu_sc as plsc

N = 10000
D = 128
H = 256
NC = 40
EPS = 1e-7

NPAD = 10112
ROWS_PT = NPAD // 16
E_PAD = 323584
CH = 64
IDX_PAD = 12288

@functools.cache
def _mesh():
    return plsc.VectorSubcoreMesh(core_axis_name="c", subcore_axis_name="s",
                                  num_cores=2, num_subcores=16)


def _zero_rows(buf, nrow, ncol):
    z = jnp.zeros((16,), jnp.float32)
    npc = ncol // 16

    def body(i, _):
        buf[i // npc, pl.ds((i % npc) * 16, 16)] = z
        return 0

    lax.fori_loop(0, nrow * npc, body, 0)


def _zero_acc_slice(acc, rows, base, ncol):
    _zero_rows(rows, CH, ncol)
    for j in range(ROWS_PT // CH):
        pltpu.sync_copy(rows, acc.at[pl.ds(base + j * CH, CH)])
    rem = ROWS_PT % CH
    if rem:
        pltpu.sync_copy(rows.at[pl.ds(0, rem)],
                        acc.at[pl.ds(base + (ROWS_PT // CH) * CH, rem)])


def _sc_emb_deg(idx_hbm, emb_hbm, dst_hbm, ea_hbm, h_out, deg_out,
                idxv, didx0, eav0, rows0, didx1, eav1, rows1, acc,
                sem, ssem0, ssem1):
    c = lax.axis_index("c")
    s = lax.axis_index("s")
    w = s * 2 + c
    rpw = IDX_PAD // 32
    for j in range(rpw // CH):
        base = w * rpw + j * CH
        pltpu.sync_copy(idx_hbm.at[pl.ds(base, CH)], idxv)
        pltpu.async_copy(emb_hbm.at[idxv], rows0, sem).wait()
        pltpu.sync_copy(rows0, h_out.at[pl.ds(base, CH)])
    br = s * ROWS_PT
    _zero_acc_slice(acc, rows0, br, D)
    _zero_rows(rows1, CH, D)
    plsc.subcore_barrier()

    epw = E_PAD // 32
    nch = epw // CH
    bufs = ((didx0, eav0, rows0, ssem0), (didx1, eav1, rows1, ssem1))

    def stage(b, chunk):
        didx, eav, rows, _ = bufs[b]
        base = w * epw + chunk * CH
        pltpu.sync_copy(dst_hbm.at[pl.ds(base, CH)], didx)
        pltpu.sync_copy(ea_hbm.at[pl.ds(base, CH)], eav)

        def sb(g, _):
            wv = eav[pl.ds(g * 16, 16)]
            for l in range(16):
                rows[g * 16 + l, pl.ds(0, 16)] = jnp.full(
                    (16,), wv[l], jnp.float32)
            return 0

        lax.fori_loop(0, CH // 16, sb, 0)

    stage(0, 0)

    def body(i, _):
        for b in range(2):
            cur = 2 * i + b
            didx, eav, rows, ssem = bufs[b]
            odidx, _, orows, ossem = bufs[1 - b]
            pltpu.async_copy(rows, acc.at[didx], ssem, add=True)

            @pl.when(cur + 1 < nch)
            def _():
                @pl.when(cur >= 1)
                def _():
                    pltpu.make_async_copy(orows, acc.at[odidx], ossem).wait()

                stage(1 - b, cur + 1)
        return 0

    lax.fori_loop(0, nch // 2, body, 0)
    pltpu.make_async_copy(rows0, acc.at[didx0], ssem0).wait()
    pltpu.make_async_copy(rows1, acc.at[didx1], ssem1).wait()
    plsc.subcore_barrier()
    pltpu.sync_copy(acc.at[pl.ds(br, ROWS_PT)],
                    deg_out.at[pl.ds(c * NPAD + br, ROWS_PT)])


def _emb_deg_call(idxp, emb, dstp, eap):
    return pl.kernel(
        _sc_emb_deg,
        out_type=[
            jax.ShapeDtypeStruct((IDX_PAD, D), jnp.float32),
            jax.ShapeDtypeStruct((2 * NPAD, D), jnp.float32),
        ],
        mesh=_mesh(),
        scratch_types=[
            pltpu.VMEM((CH,), jnp.int32),
            pltpu.VMEM((CH,), jnp.int32),
            pltpu.VMEM((CH,), jnp.float32),
            pltpu.VMEM((CH, D), jnp.float32),
            pltpu.VMEM((CH,), jnp.int32),
            pltpu.VMEM((CH,), jnp.float32),
            pltpu.VMEM((CH, D), jnp.float32),
            pltpu.VMEM_SHARED((NPAD, D), jnp.float32),
            pltpu.SemaphoreType.DMA,
            pltpu.SemaphoreType.DMA,
            pltpu.SemaphoreType.DMA,
        ],
    )(idxp, emb, dstp, eap)


def _sc_edge(src_hbm, dst_hbm, p_hbm, out_hbm,
             sidx0, didx0, rows0, sidx1, didx1, rows1,
             sidx2, didx2, rows2, sidx3, didx3, rows3, acc,
             gsem0, gsem1, gsem2, gsem3, ssem0, ssem1, ssem2, ssem3):
    c = lax.axis_index("c")
    s = lax.axis_index("s")
    br = s * ROWS_PT
    _zero_acc_slice(acc, rows0, br, D)
    plsc.subcore_barrier()

    coff = c * N
    ept = E_PAD // 16
    nch = ept // CH
    bufs = ((sidx0, didx0, rows0, gsem0, ssem0),
            (sidx1, didx1, rows1, gsem1, ssem1),
            (sidx2, didx2, rows2, gsem2, ssem2),
            (sidx3, didx3, rows3, gsem3, ssem3))

    def stage(b, chunk):
        sidx, didx, rows, gsem, _ = bufs[b]
        base = s * ept + chunk * CH
        pltpu.sync_copy(src_hbm.at[pl.ds(base, CH)], sidx)
        pltpu.sync_copy(dst_hbm.at[pl.ds(base, CH)], didx)
        for k in range(CH // 16):
            sidx[pl.ds(k * 16, 16)] = sidx[pl.ds(k * 16, 16)] + coff
        pltpu.async_copy(p_hbm.at[sidx], rows, gsem)

    stage(0, 0)
    stage(1, 1)

    def body(i, _):
        for b in range(4):
            cur = 4 * i + b

            @pl.when(cur < nch)
            def _():
                sidx, didx, rows, gsem, ssem = bufs[b]
                nb = (b + 2) % 4
                _, ndidx, nrows, _, nssem = bufs[nb]

                @pl.when(cur + 2 < nch)
                def _():
                    @pl.when(cur >= 2)
                    def _():
                        pltpu.make_async_copy(
                            nrows, acc.at[ndidx], nssem).wait()

                    stage(nb, cur + 2)

                pltpu.make_async_copy(p_hbm.at[sidx], rows, gsem).wait()
                pltpu.async_copy(rows, acc.at[didx], ssem, add=True)
        return 0

    lax.fori_loop(0, (nch + 3) // 4, body, 0)
    pltpu.make_async_copy(rows0, acc.at[didx0], ssem0).wait()
    pltpu.make_async_copy(rows1, acc.at[didx1], ssem1).wait()
    pltpu.make_async_copy(rows2, acc.at[didx2], ssem2).wait()
    pltpu.make_async_copy(rows3, acc.at[didx3], ssem3).wait()
    plsc.subcore_barrier()
    pltpu.sync_copy(acc.at[pl.ds(br, ROWS_PT)],
                    out_hbm.at[pl.ds(c * NPAD + br, ROWS_PT)])


def _edge_call(srcp, dstp, p2):
    return pl.kernel(
        _sc_edge,
        out_type=jax.ShapeDtypeStruct((2 * NPAD, D), jnp.float32),
        mesh=_mesh(),
        scratch_types=(
            [pltpu.VMEM((CH,), jnp.int32),
             pltpu.VMEM((CH,), jnp.int32),
             pltpu.VMEM((CH, D), jnp.float32)] * 4
            + [pltpu.VMEM_SHARED((NPAD, D), jnp.float32)]
            + [pltpu.SemaphoreType.DMA] * 8),
    )(srcp, dstp, p2)


def _sc_gcn(src_hbm, dst_hbm, ea_hbm, q_hbm, out_hbm,
            sidx0, didx0, eav0, rows0, sidx1, didx1, eav1, rows1, acc,
            gsem0, gsem1, ssem0, ssem1):
    c = lax.axis_index("c")
    s = lax.axis_index("s")
    w = s * 2 + c
    br = s * ROWS_PT
    _zero_acc_slice(acc, rows0, br, D)
    plsc.subcore_barrier()

    epw = E_PAD // 32
    nch = epw // CH
    bufs = ((sidx0, didx0, eav0, rows0, gsem0, ssem0),
            (sidx1, didx1, eav1, rows1, gsem1, ssem1))

    def stage(b, chunk):
        sidx, didx, eav, rows, gsem, _ = bufs[b]
        base = w * epw + chunk * CH
        pltpu.sync_copy(src_hbm.at[pl.ds(base, CH)], sidx)
        pltpu.sync_copy(dst_hbm.at[pl.ds(base, CH)], didx)
        pltpu.sync_copy(ea_hbm.at[pl.ds(base, CH)], eav)
        pltpu.async_copy(q_hbm.at[sidx], rows, gsem)

    def finish(b):
        sidx, didx, eav, rows, gsem, ssem = bufs[b]
        pltpu.make_async_copy(q_hbm.at[sidx], rows, gsem).wait()

        def mb(g, _):
            wv = eav[pl.ds(g * 16, 16)]
            for l in range(16):
                wgt = wv[l]
                k = g * 16 + l
                for j in range(D // 16):
                    rows[k, pl.ds(j * 16, 16)] = (
                        rows[k, pl.ds(j * 16, 16)] * wgt)
            return 0

        lax.fori_loop(0, CH // 16, mb, 0)
        pltpu.async_copy(rows, acc.at[didx], ssem, add=True)

    stage(0, 0)

    def body(i, _):
        for b in range(2):
            cur = 2 * i + b
            _, odidx, _, orows, _, ossem = bufs[1 - b]

            @pl.when(cur + 1 < nch)
            def _():
                @pl.when(cur >= 1)
                def _():
                    pltpu.make_async_copy(orows, acc.at[odidx], ossem).wait()

                stage(1 - b, cur + 1)

            finish(b)
        return 0

    lax.fori_loop(0, nch // 2, body, 0)
    pltpu.make_async_copy(rows0, acc.at[didx0], ssem0).wait()
    pltpu.make_async_copy(rows1, acc.at[didx1], ssem1).wait()
    plsc.subcore_barrier()
    pltpu.sync_copy(acc.at[pl.ds(br, ROWS_PT)],
                    out_hbm.at[pl.ds(c * NPAD + br, ROWS_PT)])


def _gcn_call(srcp, dstp, eap, q):
    return pl.kernel(
        _sc_gcn,
        out_type=jax.ShapeDtypeStruct((2 * NPAD, D), jnp.float32),
        mesh=_mesh(),
        scratch_types=[
            pltpu.VMEM((CH,), jnp.int32),
            pltpu.VMEM((CH,), jnp.int32),
            pltpu.VMEM((CH,), jnp.float32),
            pltpu.VMEM((CH, D), jnp.float32),
            pltpu.VMEM((CH,), jnp.int32),
            pltpu.VMEM((CH,), jnp.int32),
            pltpu.VMEM((CH,), jnp.float32),
            pltpu.VMEM((CH, D), jnp.float32),
            pltpu.VMEM_SHARED((NPAD, D), jnp.float32),
            pltpu.SemaphoreType.DMA,
            pltpu.SemaphoreType.DMA,
            pltpu.SemaphoreType.DMA,
            pltpu.SemaphoreType.DMA,
        ],
    )(srcp, dstp, eap, q)


_BR = 1000


def _ln(v, g, b):
    mu = jnp.mean(v, axis=-1, keepdims=True)
    var = jnp.mean((v - mu) * (v - mu), axis=-1, keepdims=True)
    return (v - mu) * lax.rsqrt(var + 1e-5) * g + b


def _tc_prep(h_ref, g_ref, b_ref, t_ref, r_ref, p_ref):
    h = h_ref[...]
    r = jnp.maximum(_ln(h, g_ref[...], b_ref[...]), 0.0)
    m = r + EPS
    ex = jnp.exp(t_ref[...] * m)
    r_ref[...] = r
    p_ref[0, :, :] = ex
    p_ref[1, :, :] = ex * m


def _prep_call(h, g, b, trow):
    row = lambda i: (i, 0)
    one = lambda i: (0, 0)
    return pl.pallas_call(
        _tc_prep,
        grid=(N // _BR,),
        in_specs=[
            pl.BlockSpec((_BR, D), row),
            pl.BlockSpec((1, D), one),
            pl.BlockSpec((1, D), one),
            pl.BlockSpec((1, D), one),
        ],
        out_specs=[
            pl.BlockSpec((_BR, D), row),
            pl.BlockSpec((2, _BR, D), lambda i: (0, i, 0)),
        ],
        out_shape=[
            jax.ShapeDtypeStruct((N, D), jnp.float32),
            jax.ShapeDtypeStruct((2, N, D), jnp.float32),
        ],
    )(h, g, b, trow)


def _tc_mlp(h_ref, r_ref, den_ref, num_ref, w1_ref, b1_ref, lg_ref, lb_ref,
            w2_ref, b2_ref, h2_ref):
    aggr = num_ref[...] / (den_ref[...] + 1e-16)
    out = aggr + r_ref[...]
    z = jnp.dot(out, w1_ref[...], preferred_element_type=jnp.float32)
    z = _ln(z + b1_ref[...], lg_ref[...], lb_ref[...])
    z = jnp.maximum(z, 0.0)
    z2 = jnp.dot(z, w2_ref[...], preferred_element_type=jnp.float32)
    h2_ref[...] = h_ref[...] + z2 + b2_ref[...]


def _mlp_call(h, r, den, num, w1, b1, lg, lb, w2, b2):
    row = lambda i: (i, 0)
    one = lambda i: (0, 0)
    return pl.pallas_call(
        _tc_mlp,
        grid=(N // _BR,),
        in_specs=[
            pl.BlockSpec((_BR, D), row),
            pl.BlockSpec((_BR, D), row),
            pl.BlockSpec((_BR, D), row),
            pl.BlockSpec((_BR, D), row),
            pl.BlockSpec((D, H), one),
            pl.BlockSpec((1, H), one),
            pl.BlockSpec((1, H), one),
            pl.BlockSpec((1, H), one),
            pl.BlockSpec((H, D), one),
            pl.BlockSpec((1, D), one),
        ],
        out_specs=pl.BlockSpec((_BR, D), row),
        out_shape=jax.ShapeDtypeStruct((N, D), jnp.float32),
    )(h, r, den, num, w1, b1, lg, lb, w2, b2)


def _tc_degsum(d_ref, o_ref):
    d = d_ref[...]
    o_ref[...] = d[0:NPAD, 0:1] + d[NPAD:2 * NPAD, 0:1] + 1.0


def _degsum_call(deg2):
    return pl.pallas_call(
        _tc_degsum,
        out_shape=jax.ShapeDtypeStruct((NPAD, 1), jnp.float32),
    )(deg2)


def _tc_gcnprep(h_ref, wg_ref, deg_ref, q_ref, dis_ref, st_ref):
    hw = jnp.dot(h_ref[...], wg_ref[...], preferred_element_type=jnp.float32)
    deg = deg_ref[...]
    dis = jnp.where(deg > 0, 1.0 / jnp.sqrt(deg), 0.0)
    q_ref[...] = hw * dis
    dis_ref[...] = dis
    st_ref[...] = hw * (dis * dis)


def _gcnprep_call(h, wgp, degcol):
    row = lambda i: (i, 0)
    one = lambda i: (0, 0)
    return pl.pallas_call(
        _tc_gcnprep,
        grid=(N // _BR,),
        in_specs=[
            pl.BlockSpec((_BR, D), row),
            pl.BlockSpec((D, D), one),
            pl.BlockSpec((_BR, 1), row),
        ],
        out_specs=[
            pl.BlockSpec((_BR, D), row),
            pl.BlockSpec((_BR, 1), row),
            pl.BlockSpec((_BR, D), row),
        ],
        out_shape=[
            jax.ShapeDtypeStruct((N, D), jnp.float32),
            jax.ShapeDtypeStruct((N, 1), jnp.float32),
            jax.ShapeDtypeStruct((N, D), jnp.float32),
        ],
    )(h, wgp, degcol)


def _tc_final(e0_ref, e1_ref, dis_ref, st_ref, bg_ref, o_ref):
    o = dis_ref[...] * (e0_ref[...] + e1_ref[...]) + st_ref[...] + bg_ref[...]
    mx = jnp.max(o, axis=-1, keepdims=True)
    lse = jnp.log(jnp.sum(jnp.exp(o - mx), axis=-1, keepdims=True))
    o_ref[...] = o - mx - lse


def _final_call(e0, e1, dis, st, bgp):
    row = lambda i: (i, 0)
    one = lambda i: (0, 0)
    return pl.pallas_call(
        _tc_final,
        grid=(N // _BR,),
        in_specs=[
            pl.BlockSpec((_BR, D), row),
            pl.BlockSpec((_BR, D), row),
            pl.BlockSpec((_BR, 1), row),
            pl.BlockSpec((_BR, D), row),
            pl.BlockSpec((1, D), one),
        ],
        out_specs=pl.BlockSpec((_BR, D), row),
        out_shape=jax.ShapeDtypeStruct((N, D), jnp.float32),
    )(e0, e1, dis, st, bgp)


def kernel(x, edge_index, edge_attr, emb, ln_g, ln_b, t, W1, b1,
           mlp_ln_g, mlp_ln_b, W2, b2, Wg, bg):
    f32 = jnp.float32
    src = edge_index[0]
    dst = edge_index[1]
    e = src.shape[0]
    npad_e = E_PAD - e
    srcp = jnp.concatenate([src, jnp.zeros((npad_e,), src.dtype)])
    dstp = jnp.concatenate([dst, jnp.full((npad_e,), N, dst.dtype)])
    eap = jnp.concatenate([edge_attr, jnp.zeros((npad_e,), f32)])
    idxp = jnp.concatenate(
        [x[:, 0].astype(jnp.int32),
         jnp.zeros((IDX_PAD - x.shape[0],), jnp.int32)])

    hfull, deg2 = _emb_deg_call(idxp, emb, dstp, eap)
    h = hfull[:N]
    degcol = _degsum_call(deg2)[:N]

    for i in range(2):
        trow = jnp.full((1, D), t[i], f32)
        r, p2 = _prep_call(h, ln_g[i].reshape(1, D), ln_b[i].reshape(1, D),
                           trow)
        s2 = _edge_call(srcp, dstp, p2.reshape(2 * N, D))
        den = s2[:N]
        num = s2[NPAD:NPAD + N]
        h = _mlp_call(h, r, den, num, W1[i], b1[i].reshape(1, H),
                      mlp_ln_g[i].reshape(1, H), mlp_ln_b[i].reshape(1, H),
                      W2[i], b2[i].reshape(1, D))

    wgp = jnp.concatenate([Wg, jnp.zeros((D, D - NC), f32)], axis=1)
    q, dis, st = _gcnprep_call(h, wgp, degcol)
    eacc = _gcn_call(srcp, dstp, eap, q)
    e0 = eacc[:N]
    e1 = eacc[NPAD:NPAD + N]
    bgp = jnp.concatenate([bg, jnp.full((D - NC,), -1e30, f32)])
    out = _final_call(e0, e1, dis, st, bgp.reshape(1, D))
    return out[:, :NC]


# speedup vs baseline: 1.0758x; 1.0758x over previous
import functools

import jax
import jax.numpy as jnp
from jax import lax
from jax.experimental import pallas as pl
from jax.experimental.pallas import tpu as pltpu
from jax.experimental.pallas import tpu_sc as plsc

N = 10000
D = 128
H = 256
NC = 40
EPS = 1e-7

NPAD = 10112
ROWS_PT = NPAD // 16
E_PAD = 323584
CH = 128
IDX_PAD = 12288

@functools.cache
def _mesh():
    return plsc.VectorSubcoreMesh(core_axis_name="c", subcore_axis_name="s",
                                  num_cores=2, num_subcores=16)


def _zero_rows(buf, nrow, ncol):
    z = jnp.zeros((16,), jnp.float32)
    npc = ncol // 16

    def body(i, _):
        buf[i // npc, pl.ds((i % npc) * 16, 16)] = z
        return 0

    lax.fori_loop(0, nrow * npc, body, 0)


def _zero_acc_slice(acc, rows, base, ncol):
    _zero_rows(rows, CH, ncol)
    for j in range(ROWS_PT // CH):
        pltpu.sync_copy(rows, acc.at[pl.ds(base + j * CH, CH)])
    rem = ROWS_PT % CH
    if rem:
        pltpu.sync_copy(rows.at[pl.ds(0, rem)],
                        acc.at[pl.ds(base + (ROWS_PT // CH) * CH, rem)])


def _sc_emb_deg(idx_hbm, emb_hbm, dst_hbm, ea_hbm, h_out, deg_out,
                idxv, didx0, eav0, rows0, didx1, eav1, rows1, acc,
                sem, ssem0, ssem1):
    c = lax.axis_index("c")
    s = lax.axis_index("s")
    w = s * 2 + c
    rpw = IDX_PAD // 32
    for j in range(rpw // CH):
        base = w * rpw + j * CH
        pltpu.sync_copy(idx_hbm.at[pl.ds(base, CH)], idxv)
        pltpu.async_copy(emb_hbm.at[idxv], rows0, sem).wait()
        pltpu.sync_copy(rows0, h_out.at[pl.ds(base, CH)])
    br = s * ROWS_PT
    _zero_acc_slice(acc, rows0, br, D)
    _zero_rows(rows1, CH, D)
    plsc.subcore_barrier()

    epw = E_PAD // 32
    nch = epw // CH
    bufs = ((didx0, eav0, rows0, ssem0), (didx1, eav1, rows1, ssem1))

    def stage(b, chunk):
        didx, eav, rows, _ = bufs[b]
        base = w * epw + chunk * CH
        pltpu.sync_copy(dst_hbm.at[pl.ds(base, CH)], didx)
        pltpu.sync_copy(ea_hbm.at[pl.ds(base, CH)], eav)

        def sb(g, _):
            wv = eav[pl.ds(g * 16, 16)]
            for l in range(16):
                rows[g * 16 + l, pl.ds(0, 16)] = jnp.full(
                    (16,), wv[l], jnp.float32)
            return 0

        lax.fori_loop(0, CH // 16, sb, 0)

    stage(0, 0)

    def body(i, _):
        for b in range(2):
            cur = 2 * i + b
            didx, eav, rows, ssem = bufs[b]
            odidx, _, orows, ossem = bufs[1 - b]
            pltpu.async_copy(rows, acc.at[didx], ssem, add=True)

            @pl.when(cur + 1 < nch)
            def _():
                @pl.when(cur >= 1)
                def _():
                    pltpu.make_async_copy(orows, acc.at[odidx], ossem).wait()

                stage(1 - b, cur + 1)
        return 0

    lax.fori_loop(0, nch // 2, body, 0)
    pltpu.async_copy(rows0, acc.at[didx0], ssem0, add=True)
    pltpu.make_async_copy(rows1, acc.at[didx1], ssem1).wait()
    pltpu.make_async_copy(rows0, acc.at[didx0], ssem0).wait()
    plsc.subcore_barrier()
    pltpu.sync_copy(acc.at[pl.ds(br, ROWS_PT)],
                    deg_out.at[pl.ds(c * NPAD + br, ROWS_PT)])


def _emb_deg_call(idxp, emb, dstp, eap):
    return pl.kernel(
        _sc_emb_deg,
        out_type=[
            jax.ShapeDtypeStruct((IDX_PAD, D), jnp.float32),
            jax.ShapeDtypeStruct((2 * NPAD, D), jnp.float32),
        ],
        mesh=_mesh(),
        scratch_types=[
            pltpu.VMEM((CH,), jnp.int32),
            pltpu.VMEM((CH,), jnp.int32),
            pltpu.VMEM((CH,), jnp.float32),
            pltpu.VMEM((CH, D), jnp.float32),
            pltpu.VMEM((CH,), jnp.int32),
            pltpu.VMEM((CH,), jnp.float32),
            pltpu.VMEM((CH, D), jnp.float32),
            pltpu.VMEM_SHARED((NPAD, D), jnp.float32),
            pltpu.SemaphoreType.DMA,
            pltpu.SemaphoreType.DMA,
            pltpu.SemaphoreType.DMA,
        ],
    )(idxp, emb, dstp, eap)


def _sc_edge(src_hbm, dst_hbm, p_hbm, out_hbm,
             sidx0, didx0, rows0, sidx1, didx1, rows1, acc,
             gsem0, gsem1, ssem0, ssem1):
    c = lax.axis_index("c")
    s = lax.axis_index("s")
    br = s * ROWS_PT
    _zero_acc_slice(acc, rows0, br, D)
    plsc.subcore_barrier()

    coff = c * N
    ept = E_PAD // 16
    nch = ept // CH
    bufs = ((sidx0, didx0, rows0, gsem0, ssem0),
            (sidx1, didx1, rows1, gsem1, ssem1))

    def stage(b, chunk):
        sidx, didx, rows, gsem, _ = bufs[b]
        base = s * ept + chunk * CH
        pltpu.sync_copy(src_hbm.at[pl.ds(base, CH)], sidx)
        pltpu.sync_copy(dst_hbm.at[pl.ds(base, CH)], didx)
        for k in range(CH // 16):
            sidx[pl.ds(k * 16, 16)] = sidx[pl.ds(k * 16, 16)] + coff
        pltpu.async_copy(p_hbm.at[sidx], rows, gsem)

    stage(0, 0)

    def body(i, _):
        for b in range(2):
            cur = 2 * i + b
            sidx, didx, rows, gsem, ssem = bufs[b]
            _, odidx, orows, _, ossem = bufs[1 - b]

            @pl.when(cur + 1 < nch)
            def _():
                @pl.when(cur >= 1)
                def _():
                    pltpu.make_async_copy(orows, acc.at[odidx], ossem).wait()

                stage(1 - b, cur + 1)

            pltpu.make_async_copy(p_hbm.at[sidx], rows, gsem).wait()
            pltpu.async_copy(rows, acc.at[didx], ssem, add=True)
        return 0

    lax.fori_loop(0, nch // 2, body, 0)
    pltpu.make_async_copy(rows0, acc.at[didx0], ssem0).wait()
    pltpu.make_async_copy(rows1, acc.at[didx1], ssem1).wait()
    plsc.subcore_barrier()
    pltpu.sync_copy(acc.at[pl.ds(br, ROWS_PT)],
                    out_hbm.at[pl.ds(c * NPAD + br, ROWS_PT)])


def _edge_call(srcp, dstp, p2):
    return pl.kernel(
        _sc_edge,
        out_type=jax.ShapeDtypeStruct((2 * NPAD, D), jnp.float32),
        mesh=_mesh(),
        scratch_types=(
            [pltpu.VMEM((CH,), jnp.int32),
             pltpu.VMEM((CH,), jnp.int32),
             pltpu.VMEM((CH, D), jnp.float32)] * 2
            + [pltpu.VMEM_SHARED((NPAD, D), jnp.float32)]
            + [pltpu.SemaphoreType.DMA] * 4),
    )(srcp, dstp, p2)


def _sc_gcn(src_hbm, dst_hbm, ea_hbm, q_hbm, out_hbm,
            sidx0, didx0, eav0, rows0, sidx1, didx1, eav1, rows1, acc,
            gsem0, gsem1, ssem0, ssem1):
    c = lax.axis_index("c")
    s = lax.axis_index("s")
    w = s * 2 + c
    br = s * ROWS_PT
    _zero_acc_slice(acc, rows0, br, D)
    plsc.subcore_barrier()

    epw = E_PAD // 32
    nch = epw // CH
    bufs = ((sidx0, didx0, eav0, rows0, gsem0, ssem0),
            (sidx1, didx1, eav1, rows1, gsem1, ssem1))

    def stage(b, chunk):
        sidx, didx, eav, rows, gsem, _ = bufs[b]
        base = w * epw + chunk * CH
        pltpu.sync_copy(src_hbm.at[pl.ds(base, CH)], sidx)
        pltpu.sync_copy(dst_hbm.at[pl.ds(base, CH)], didx)
        pltpu.sync_copy(ea_hbm.at[pl.ds(base, CH)], eav)
        pltpu.async_copy(q_hbm.at[sidx], rows, gsem)

    def finish(b):
        sidx, didx, eav, rows, gsem, ssem = bufs[b]
        pltpu.make_async_copy(q_hbm.at[sidx], rows, gsem).wait()

        def mb(g, _):
            wv = eav[pl.ds(g * 16, 16)]
            for l in range(16):
                wgt = wv[l]
                k = g * 16 + l
                for j in range(D // 16):
                    rows[k, pl.ds(j * 16, 16)] = (
                        rows[k, pl.ds(j * 16, 16)] * wgt)
            return 0

        lax.fori_loop(0, CH // 16, mb, 0)
        pltpu.async_copy(rows, acc.at[didx], ssem, add=True)

    stage(0, 0)

    def body(i, _):
        for b in range(2):
            cur = 2 * i + b
            _, odidx, _, orows, _, ossem = bufs[1 - b]

            @pl.when(cur + 1 < nch)
            def _():
                @pl.when(cur >= 1)
                def _():
                    pltpu.make_async_copy(orows, acc.at[odidx], ossem).wait()

                stage(1 - b, cur + 1)

            finish(b)
        return 0

    lax.fori_loop(0, nch // 2, body, 0)
    finish(0)
    pltpu.make_async_copy(rows1, acc.at[didx1], ssem1).wait()
    pltpu.make_async_copy(rows0, acc.at[didx0], ssem0).wait()
    plsc.subcore_barrier()
    pltpu.sync_copy(acc.at[pl.ds(br, ROWS_PT)],
                    out_hbm.at[pl.ds(c * NPAD + br, ROWS_PT)])


def _gcn_call(srcp, dstp, eap, q):
    return pl.kernel(
        _sc_gcn,
        out_type=jax.ShapeDtypeStruct((2 * NPAD, D), jnp.float32),
        mesh=_mesh(),
        scratch_types=[
            pltpu.VMEM((CH,), jnp.int32),
            pltpu.VMEM((CH,), jnp.int32),
            pltpu.VMEM((CH,), jnp.float32),
            pltpu.VMEM((CH, D), jnp.float32),
            pltpu.VMEM((CH,), jnp.int32),
            pltpu.VMEM((CH,), jnp.int32),
            pltpu.VMEM((CH,), jnp.float32),
            pltpu.VMEM((CH, D), jnp.float32),
            pltpu.VMEM_SHARED((NPAD, D), jnp.float32),
            pltpu.SemaphoreType.DMA,
            pltpu.SemaphoreType.DMA,
            pltpu.SemaphoreType.DMA,
            pltpu.SemaphoreType.DMA,
        ],
    )(srcp, dstp, eap, q)


_BR = 1000


def _ln(v, g, b):
    mu = jnp.mean(v, axis=-1, keepdims=True)
    var = jnp.mean((v - mu) * (v - mu), axis=-1, keepdims=True)
    return (v - mu) * lax.rsqrt(var + 1e-5) * g + b


def _tc_prep(h_ref, g_ref, b_ref, t_ref, r_ref, p_ref):
    h = h_ref[...]
    r = jnp.maximum(_ln(h, g_ref[...], b_ref[...]), 0.0)
    m = r + EPS
    ex = jnp.exp(t_ref[...] * m)
    r_ref[...] = r
    p_ref[0, :, :] = ex
    p_ref[1, :, :] = ex * m


def _prep_call(h, g, b, trow):
    row = lambda i: (i, 0)
    one = lambda i: (0, 0)
    return pl.pallas_call(
        _tc_prep,
        grid=(N // _BR,),
        in_specs=[
            pl.BlockSpec((_BR, D), row),
            pl.BlockSpec((1, D), one),
            pl.BlockSpec((1, D), one),
            pl.BlockSpec((1, D), one),
        ],
        out_specs=[
            pl.BlockSpec((_BR, D), row),
            pl.BlockSpec((2, _BR, D), lambda i: (0, i, 0)),
        ],
        out_shape=[
            jax.ShapeDtypeStruct((N, D), jnp.float32),
            jax.ShapeDtypeStruct((2, N, D), jnp.float32),
        ],
    )(h, g, b, trow)


def _tc_mlp(h_ref, r_ref, den_ref, num_ref, w1_ref, b1_ref, lg_ref, lb_ref,
            w2_ref, b2_ref, h2_ref):
    aggr = num_ref[...] / (den_ref[...] + 1e-16)
    out = aggr + r_ref[...]
    z = jnp.dot(out, w1_ref[...], preferred_element_type=jnp.float32)
    z = _ln(z + b1_ref[...], lg_ref[...], lb_ref[...])
    z = jnp.maximum(z, 0.0)
    z2 = jnp.dot(z, w2_ref[...], preferred_element_type=jnp.float32)
    h2_ref[...] = h_ref[...] + z2 + b2_ref[...]


def _mlp_call(h, r, den, num, w1, b1, lg, lb, w2, b2):
    row = lambda i: (i, 0)
    one = lambda i: (0, 0)
    return pl.pallas_call(
        _tc_mlp,
        grid=(N // _BR,),
        in_specs=[
            pl.BlockSpec((_BR, D), row),
            pl.BlockSpec((_BR, D), row),
            pl.BlockSpec((_BR, D), row),
            pl.BlockSpec((_BR, D), row),
            pl.BlockSpec((D, H), one),
            pl.BlockSpec((1, H), one),
            pl.BlockSpec((1, H), one),
            pl.BlockSpec((1, H), one),
            pl.BlockSpec((H, D), one),
            pl.BlockSpec((1, D), one),
        ],
        out_specs=pl.BlockSpec((_BR, D), row),
        out_shape=jax.ShapeDtypeStruct((N, D), jnp.float32),
    )(h, r, den, num, w1, b1, lg, lb, w2, b2)


def _tc_degsum(d_ref, o_ref):
    d = d_ref[...]
    o_ref[...] = d[0:NPAD, 0:1] + d[NPAD:2 * NPAD, 0:1] + 1.0


def _degsum_call(deg2):
    return pl.pallas_call(
        _tc_degsum,
        out_shape=jax.ShapeDtypeStruct((NPAD, 1), jnp.float32),
    )(deg2)


def _tc_gcnprep(h_ref, wg_ref, deg_ref, q_ref, dis_ref, st_ref):
    hw = jnp.dot(h_ref[...], wg_ref[...], preferred_element_type=jnp.float32)
    deg = deg_ref[...]
    dis = jnp.where(deg > 0, 1.0 / jnp.sqrt(deg), 0.0)
    q_ref[...] = hw * dis
    dis_ref[...] = dis
    st_ref[...] = hw * (dis * dis)


def _gcnprep_call(h, wgp, degcol):
    row = lambda i: (i, 0)
    one = lambda i: (0, 0)
    return pl.pallas_call(
        _tc_gcnprep,
        grid=(N // _BR,),
        in_specs=[
            pl.BlockSpec((_BR, D), row),
            pl.BlockSpec((D, D), one),
            pl.BlockSpec((_BR, 1), row),
        ],
        out_specs=[
            pl.BlockSpec((_BR, D), row),
            pl.BlockSpec((_BR, 1), row),
            pl.BlockSpec((_BR, D), row),
        ],
        out_shape=[
            jax.ShapeDtypeStruct((N, D), jnp.float32),
            jax.ShapeDtypeStruct((N, 1), jnp.float32),
            jax.ShapeDtypeStruct((N, D), jnp.float32),
        ],
    )(h, wgp, degcol)


def _tc_final(e0_ref, e1_ref, dis_ref, st_ref, bg_ref, o_ref):
    o = dis_ref[...] * (e0_ref[...] + e1_ref[...]) + st_ref[...] + bg_ref[...]
    mx = jnp.max(o, axis=-1, keepdims=True)
    lse = jnp.log(jnp.sum(jnp.exp(o - mx), axis=-1, keepdims=True))
    o_ref[...] = o - mx - lse


def _final_call(e0, e1, dis, st, bgp):
    row = lambda i: (i, 0)
    one = lambda i: (0, 0)
    return pl.pallas_call(
        _tc_final,
        grid=(N // _BR,),
        in_specs=[
            pl.BlockSpec((_BR, D), row),
            pl.BlockSpec((_BR, D), row),
            pl.BlockSpec((_BR, 1), row),
            pl.BlockSpec((_BR, D), row),
            pl.BlockSpec((1, D), one),
        ],
        out_specs=pl.BlockSpec((_BR, D), row),
        out_shape=jax.ShapeDtypeStruct((N, D), jnp.float32),
    )(e0, e1, dis, st, bgp)


def kernel(x, edge_index, edge_attr, emb, ln_g, ln_b, t, W1, b1,
           mlp_ln_g, mlp_ln_b, W2, b2, Wg, bg):
    f32 = jnp.float32
    src = edge_index[0]
    dst = edge_index[1]
    e = src.shape[0]
    npad_e = E_PAD - e
    srcp = jnp.concatenate([src, jnp.zeros((npad_e,), src.dtype)])
    dstp = jnp.concatenate([dst, jnp.full((npad_e,), N, dst.dtype)])
    eap = jnp.concatenate([edge_attr, jnp.zeros((npad_e,), f32)])
    idxp = jnp.concatenate(
        [x[:, 0].astype(jnp.int32),
         jnp.zeros((IDX_PAD - x.shape[0],), jnp.int32)])

    hfull, deg2 = _emb_deg_call(idxp, emb, dstp, eap)
    h = hfull[:N]
    degcol = _degsum_call(deg2)[:N]

    for i in range(2):
        trow = jnp.full((1, D), t[i], f32)
        r, p2 = _prep_call(h, ln_g[i].reshape(1, D), ln_b[i].reshape(1, D),
                           trow)
        s2 = _edge_call(srcp, dstp, p2.reshape(2 * N, D))
        den = s2[:N]
        num = s2[NPAD:NPAD + N]
        h = _mlp_call(h, r, den, num, W1[i], b1[i].reshape(1, H),
                      mlp_ln_g[i].reshape(1, H), mlp_ln_b[i].reshape(1, H),
                      W2[i], b2[i].reshape(1, D))

    wgp = jnp.concatenate([Wg, jnp.zeros((D, D - NC), f32)], axis=1)
    q, dis, st = _gcnprep_call(h, wgp, degcol)
    eacc = _gcn_call(srcp, dstp, eap, q)
    e0 = eacc[:N]
    e1 = eacc[NPAD:NPAD + N]
    bgp = jnp.concatenate([bg, jnp.full((D - NC,), -1e30, f32)])
    out = _final_call(e0, e1, dis, st, bgp.reshape(1, D))
    return out[:, :NC]


# speedup vs baseline: 1.0890x; 1.0123x over previous
import functools

import jax
import jax.numpy as jnp
from jax import lax
from jax.experimental import pallas as pl
from jax.experimental.pallas import tpu as pltpu
from jax.experimental.pallas import tpu_sc as plsc

N = 10000
D = 128
H = 256
NC = 40
EPS = 1e-7

NPAD = 10112
ROWS_PT = NPAD // 16
E_PAD = 323584
CH = 128
IDX_PAD = 12288

@functools.cache
def _mesh():
    return plsc.VectorSubcoreMesh(core_axis_name="c", subcore_axis_name="s",
                                  num_cores=2, num_subcores=16)


def _zero_rows(buf, nrow, ncol):
    z = jnp.zeros((16,), jnp.float32)
    npc = ncol // 16

    def body(i, _):
        buf[i // npc, pl.ds((i % npc) * 16, 16)] = z
        return 0

    lax.fori_loop(0, nrow * npc, body, 0)


def _zero_acc_slice(acc, rows, base, ncol):
    _zero_rows(rows, CH, ncol)
    for j in range(ROWS_PT // CH):
        pltpu.sync_copy(rows, acc.at[pl.ds(base + j * CH, CH)])
    rem = ROWS_PT % CH
    if rem:
        pltpu.sync_copy(rows.at[pl.ds(0, rem)],
                        acc.at[pl.ds(base + (ROWS_PT // CH) * CH, rem)])


def _sc_emb_deg(idx_hbm, emb_hbm, dst_hbm, ea_hbm, h_out, deg_out,
                idxv, didx0, eav0, rows0, didx1, eav1, rows1, acc,
                sem, ssem0, ssem1):
    c = lax.axis_index("c")
    s = lax.axis_index("s")
    w = s * 2 + c
    rpw = IDX_PAD // 32
    for j in range(rpw // CH):
        base = w * rpw + j * CH
        pltpu.sync_copy(idx_hbm.at[pl.ds(base, CH)], idxv)
        pltpu.async_copy(emb_hbm.at[idxv], rows0, sem).wait()
        pltpu.sync_copy(rows0, h_out.at[pl.ds(base, CH)])
    br = s * ROWS_PT
    _zero_acc_slice(acc, rows0, br, D)
    _zero_rows(rows1, CH, D)
    plsc.subcore_barrier()

    epw = E_PAD // 32
    nch = epw // CH
    bufs = ((didx0, eav0, rows0, ssem0), (didx1, eav1, rows1, ssem1))

    def stage(b, chunk):
        didx, eav, rows, _ = bufs[b]
        base = w * epw + chunk * CH
        pltpu.sync_copy(dst_hbm.at[pl.ds(base, CH)], didx)
        pltpu.sync_copy(ea_hbm.at[pl.ds(base, CH)], eav)

        def sb(g, _):
            wv = eav[pl.ds(g * 16, 16)]
            for l in range(16):
                rows[g * 16 + l, pl.ds(0, 16)] = jnp.full(
                    (16,), wv[l], jnp.float32)
            return 0

        lax.fori_loop(0, CH // 16, sb, 0)

    stage(0, 0)

    def body(i, _):
        for b in range(2):
            cur = 2 * i + b
            didx, eav, rows, ssem = bufs[b]
            odidx, _, orows, ossem = bufs[1 - b]
            pltpu.async_copy(rows, acc.at[didx], ssem, add=True)

            @pl.when(cur + 1 < nch)
            def _():
                @pl.when(cur >= 1)
                def _():
                    pltpu.make_async_copy(orows, acc.at[odidx], ossem).wait()

                stage(1 - b, cur + 1)
        return 0

    lax.fori_loop(0, nch // 2, body, 0)
    pltpu.async_copy(rows0, acc.at[didx0], ssem0, add=True)
    pltpu.make_async_copy(rows1, acc.at[didx1], ssem1).wait()
    pltpu.make_async_copy(rows0, acc.at[didx0], ssem0).wait()
    plsc.subcore_barrier()
    pltpu.sync_copy(acc.at[pl.ds(br, ROWS_PT)],
                    deg_out.at[pl.ds(c * NPAD + br, ROWS_PT)])


def _emb_deg_call(idxp, emb, dstp, eap):
    return pl.kernel(
        _sc_emb_deg,
        out_type=[
            jax.ShapeDtypeStruct((IDX_PAD, D), jnp.float32),
            jax.ShapeDtypeStruct((2 * NPAD, D), jnp.float32),
        ],
        mesh=_mesh(),
        scratch_types=[
            pltpu.VMEM((CH,), jnp.int32),
            pltpu.VMEM((CH,), jnp.int32),
            pltpu.VMEM((CH,), jnp.float32),
            pltpu.VMEM((CH, D), jnp.float32),
            pltpu.VMEM((CH,), jnp.int32),
            pltpu.VMEM((CH,), jnp.float32),
            pltpu.VMEM((CH, D), jnp.float32),
            pltpu.VMEM_SHARED((NPAD, D), jnp.float32),
            pltpu.SemaphoreType.DMA,
            pltpu.SemaphoreType.DMA,
            pltpu.SemaphoreType.DMA,
        ],
    )(idxp, emb, dstp, eap)


def _sc_edge(src_hbm, dst_hbm, p_hbm, out_hbm,
             sidx0, didx0, rows0, sidx1, didx1, rows1, acc,
             gsem0, gsem1, ssem0, ssem1):
    c = lax.axis_index("c")
    s = lax.axis_index("s")
    br = s * ROWS_PT
    _zero_acc_slice(acc, rows0, br, D)
    plsc.subcore_barrier()

    coff = c * N
    ept = E_PAD // 16
    nch = ept // CH
    bufs = ((sidx0, didx0, rows0, gsem0, ssem0),
            (sidx1, didx1, rows1, gsem1, ssem1))

    def stage(b, chunk):
        sidx, didx, rows, gsem, _ = bufs[b]
        base = s * ept + chunk * CH
        pltpu.sync_copy(src_hbm.at[pl.ds(base, CH)], sidx)
        pltpu.sync_copy(dst_hbm.at[pl.ds(base, CH)], didx)
        for k in range(CH // 16):
            sidx[pl.ds(k * 16, 16)] = sidx[pl.ds(k * 16, 16)] + coff
        pltpu.async_copy(p_hbm.at[sidx], rows, gsem)

    stage(0, 0)

    def body(i, _):
        for b in range(2):
            cur = 2 * i + b
            sidx, didx, rows, gsem, ssem = bufs[b]
            _, odidx, orows, _, ossem = bufs[1 - b]

            @pl.when(cur + 1 < nch)
            def _():
                @pl.when(cur >= 1)
                def _():
                    pltpu.make_async_copy(orows, acc.at[odidx], ossem).wait()

                stage(1 - b, cur + 1)

            pltpu.make_async_copy(p_hbm.at[sidx], rows, gsem).wait()
            pltpu.async_copy(rows, acc.at[didx], ssem, add=True)
        return 0

    lax.fori_loop(0, nch // 2, body, 0)
    pltpu.make_async_copy(rows0, acc.at[didx0], ssem0).wait()
    pltpu.make_async_copy(rows1, acc.at[didx1], ssem1).wait()
    plsc.subcore_barrier()
    pltpu.sync_copy(acc.at[pl.ds(br, ROWS_PT)],
                    out_hbm.at[pl.ds(c * NPAD + br, ROWS_PT)])


def _edge_call(srcp, dstp, p2):
    return pl.kernel(
        _sc_edge,
        out_type=jax.ShapeDtypeStruct((2 * NPAD, D), jnp.float32),
        mesh=_mesh(),
        scratch_types=(
            [pltpu.VMEM((CH,), jnp.int32),
             pltpu.VMEM((CH,), jnp.int32),
             pltpu.VMEM((CH, D), jnp.float32)] * 2
            + [pltpu.VMEM_SHARED((NPAD, D), jnp.float32)]
            + [pltpu.SemaphoreType.DMA] * 4),
    )(srcp, dstp, p2)


def _sc_gcn(src_hbm, dst_hbm, ea_hbm, q_hbm, out_hbm,
            sidx0, didx0, eav0, rows0, sidx1, didx1, eav1, rows1, acc,
            gsem0, gsem1, ssem0, ssem1):
    c = lax.axis_index("c")
    s = lax.axis_index("s")
    w = s * 2 + c
    br = s * ROWS_PT
    _zero_acc_slice(acc, rows0, br, D)
    plsc.subcore_barrier()

    epw = E_PAD // 32
    nch = epw // CH
    bufs = ((sidx0, didx0, eav0, rows0, gsem0, ssem0),
            (sidx1, didx1, eav1, rows1, gsem1, ssem1))

    def stage(b, chunk):
        sidx, didx, eav, rows, gsem, _ = bufs[b]
        base = w * epw + chunk * CH
        pltpu.sync_copy(src_hbm.at[pl.ds(base, CH)], sidx)
        pltpu.sync_copy(dst_hbm.at[pl.ds(base, CH)], didx)
        pltpu.sync_copy(ea_hbm.at[pl.ds(base, CH)], eav)
        pltpu.async_copy(q_hbm.at[sidx], rows, gsem)

    def finish(b):
        sidx, didx, eav, rows, gsem, ssem = bufs[b]
        pltpu.make_async_copy(q_hbm.at[sidx], rows, gsem).wait()

        def mb(g, _):
            wv = eav[pl.ds(g * 16, 16)]
            for l in range(16):
                wgt = wv[l]
                k = g * 16 + l
                for j in range(3):
                    rows[k, pl.ds(j * 16, 16)] = (
                        rows[k, pl.ds(j * 16, 16)] * wgt)
            return 0

        lax.fori_loop(0, CH // 16, mb, 0)
        pltpu.async_copy(rows, acc.at[didx], ssem, add=True)

    stage(0, 0)

    def body(i, _):
        for b in range(2):
            cur = 2 * i + b
            _, odidx, _, orows, _, ossem = bufs[1 - b]

            @pl.when(cur + 1 < nch)
            def _():
                @pl.when(cur >= 1)
                def _():
                    pltpu.make_async_copy(orows, acc.at[odidx], ossem).wait()

                stage(1 - b, cur + 1)

            finish(b)
        return 0

    lax.fori_loop(0, nch // 2, body, 0)
    finish(0)
    pltpu.make_async_copy(rows1, acc.at[didx1], ssem1).wait()
    pltpu.make_async_copy(rows0, acc.at[didx0], ssem0).wait()
    plsc.subcore_barrier()
    pltpu.sync_copy(acc.at[pl.ds(br, ROWS_PT)],
                    out_hbm.at[pl.ds(c * NPAD + br, ROWS_PT)])


def _gcn_call(srcp, dstp, eap, q):
    return pl.kernel(
        _sc_gcn,
        out_type=jax.ShapeDtypeStruct((2 * NPAD, D), jnp.float32),
        mesh=_mesh(),
        scratch_types=[
            pltpu.VMEM((CH,), jnp.int32),
            pltpu.VMEM((CH,), jnp.int32),
            pltpu.VMEM((CH,), jnp.float32),
            pltpu.VMEM((CH, D), jnp.float32),
            pltpu.VMEM((CH,), jnp.int32),
            pltpu.VMEM((CH,), jnp.int32),
            pltpu.VMEM((CH,), jnp.float32),
            pltpu.VMEM((CH, D), jnp.float32),
            pltpu.VMEM_SHARED((NPAD, D), jnp.float32),
            pltpu.SemaphoreType.DMA,
            pltpu.SemaphoreType.DMA,
            pltpu.SemaphoreType.DMA,
            pltpu.SemaphoreType.DMA,
        ],
    )(srcp, dstp, eap, q)


_BR = 1000


def _ln(v, g, b):
    mu = jnp.mean(v, axis=-1, keepdims=True)
    var = jnp.mean((v - mu) * (v - mu), axis=-1, keepdims=True)
    return (v - mu) * lax.rsqrt(var + 1e-5) * g + b


def _tc_prep(h_ref, g_ref, b_ref, t_ref, r_ref, p_ref):
    h = h_ref[...]
    r = jnp.maximum(_ln(h, g_ref[...], b_ref[...]), 0.0)
    m = r + EPS
    ex = jnp.exp(t_ref[...] * m)
    r_ref[...] = r
    p_ref[0, :, :] = ex
    p_ref[1, :, :] = ex * m


def _prep_call(h, g, b, trow):
    row = lambda i: (i, 0)
    one = lambda i: (0, 0)
    return pl.pallas_call(
        _tc_prep,
        grid=(N // _BR,),
        in_specs=[
            pl.BlockSpec((_BR, D), row),
            pl.BlockSpec((1, D), one),
            pl.BlockSpec((1, D), one),
            pl.BlockSpec((1, D), one),
        ],
        out_specs=[
            pl.BlockSpec((_BR, D), row),
            pl.BlockSpec((2, _BR, D), lambda i: (0, i, 0)),
        ],
        out_shape=[
            jax.ShapeDtypeStruct((N, D), jnp.float32),
            jax.ShapeDtypeStruct((2, N, D), jnp.float32),
        ],
    )(h, g, b, trow)


def _tc_mlp(h_ref, r_ref, den_ref, num_ref, w1_ref, b1_ref, lg_ref, lb_ref,
            w2_ref, b2_ref, h2_ref):
    aggr = num_ref[...] / (den_ref[...] + 1e-16)
    out = aggr + r_ref[...]
    z = jnp.dot(out, w1_ref[...], preferred_element_type=jnp.float32)
    z = _ln(z + b1_ref[...], lg_ref[...], lb_ref[...])
    z = jnp.maximum(z, 0.0)
    z2 = jnp.dot(z, w2_ref[...], preferred_element_type=jnp.float32)
    h2_ref[...] = h_ref[...] + z2 + b2_ref[...]


def _mlp_call(h, r, den, num, w1, b1, lg, lb, w2, b2):
    row = lambda i: (i, 0)
    one = lambda i: (0, 0)
    return pl.pallas_call(
        _tc_mlp,
        grid=(N // _BR,),
        in_specs=[
            pl.BlockSpec((_BR, D), row),
            pl.BlockSpec((_BR, D), row),
            pl.BlockSpec((_BR, D), row),
            pl.BlockSpec((_BR, D), row),
            pl.BlockSpec((D, H), one),
            pl.BlockSpec((1, H), one),
            pl.BlockSpec((1, H), one),
            pl.BlockSpec((1, H), one),
            pl.BlockSpec((H, D), one),
            pl.BlockSpec((1, D), one),
        ],
        out_specs=pl.BlockSpec((_BR, D), row),
        out_shape=jax.ShapeDtypeStruct((N, D), jnp.float32),
    )(h, r, den, num, w1, b1, lg, lb, w2, b2)


def _tc_degsum(d_ref, o_ref):
    d = d_ref[...]
    o_ref[...] = d[0:NPAD, 0:1] + d[NPAD:2 * NPAD, 0:1] + 1.0


def _degsum_call(deg2):
    return pl.pallas_call(
        _tc_degsum,
        out_shape=jax.ShapeDtypeStruct((NPAD, 1), jnp.float32),
    )(deg2)


def _tc_gcnprep(h_ref, wg_ref, deg_ref, q_ref, dis_ref, st_ref):
    hw = jnp.dot(h_ref[...], wg_ref[...], preferred_element_type=jnp.float32)
    deg = deg_ref[...]
    dis = jnp.where(deg > 0, 1.0 / jnp.sqrt(deg), 0.0)
    q_ref[...] = hw * dis
    dis_ref[...] = dis
    st_ref[...] = hw * (dis * dis)


def _gcnprep_call(h, wgp, degcol):
    row = lambda i: (i, 0)
    one = lambda i: (0, 0)
    return pl.pallas_call(
        _tc_gcnprep,
        grid=(N // _BR,),
        in_specs=[
            pl.BlockSpec((_BR, D), row),
            pl.BlockSpec((D, D), one),
            pl.BlockSpec((_BR, 1), row),
        ],
        out_specs=[
            pl.BlockSpec((_BR, D), row),
            pl.BlockSpec((_BR, 1), row),
            pl.BlockSpec((_BR, D), row),
        ],
        out_shape=[
            jax.ShapeDtypeStruct((N, D), jnp.float32),
            jax.ShapeDtypeStruct((N, 1), jnp.float32),
            jax.ShapeDtypeStruct((N, D), jnp.float32),
        ],
    )(h, wgp, degcol)


def _tc_final(e0_ref, e1_ref, dis_ref, st_ref, bg_ref, o_ref):
    o = dis_ref[...] * (e0_ref[...] + e1_ref[...]) + st_ref[...] + bg_ref[...]
    mx = jnp.max(o, axis=-1, keepdims=True)
    lse = jnp.log(jnp.sum(jnp.exp(o - mx), axis=-1, keepdims=True))
    o_ref[...] = o - mx - lse


def _final_call(e0, e1, dis, st, bgp):
    row = lambda i: (i, 0)
    one = lambda i: (0, 0)
    return pl.pallas_call(
        _tc_final,
        grid=(N // _BR,),
        in_specs=[
            pl.BlockSpec((_BR, D), row),
            pl.BlockSpec((_BR, D), row),
            pl.BlockSpec((_BR, 1), row),
            pl.BlockSpec((_BR, D), row),
            pl.BlockSpec((1, D), one),
        ],
        out_specs=pl.BlockSpec((_BR, D), row),
        out_shape=jax.ShapeDtypeStruct((N, D), jnp.float32),
    )(e0, e1, dis, st, bgp)


def kernel(x, edge_index, edge_attr, emb, ln_g, ln_b, t, W1, b1,
           mlp_ln_g, mlp_ln_b, W2, b2, Wg, bg):
    f32 = jnp.float32
    src = edge_index[0]
    dst = edge_index[1]
    e = src.shape[0]
    npad_e = E_PAD - e
    srcp = jnp.concatenate([src, jnp.zeros((npad_e,), src.dtype)])
    dstp = jnp.concatenate([dst, jnp.full((npad_e,), N, dst.dtype)])
    eap = jnp.concatenate([edge_attr, jnp.zeros((npad_e,), f32)])
    idxp = jnp.concatenate(
        [x[:, 0].astype(jnp.int32),
         jnp.zeros((IDX_PAD - x.shape[0],), jnp.int32)])

    hfull, deg2 = _emb_deg_call(idxp, emb, dstp, eap)
    h = hfull[:N]
    degcol = _degsum_call(deg2)[:N]

    for i in range(2):
        trow = jnp.full((1, D), t[i], f32)
        r, p2 = _prep_call(h, ln_g[i].reshape(1, D), ln_b[i].reshape(1, D),
                           trow)
        s2 = _edge_call(srcp, dstp, p2.reshape(2 * N, D))
        den = s2[:N]
        num = s2[NPAD:NPAD + N]
        h = _mlp_call(h, r, den, num, W1[i], b1[i].reshape(1, H),
                      mlp_ln_g[i].reshape(1, H), mlp_ln_b[i].reshape(1, H),
                      W2[i], b2[i].reshape(1, D))

    wgp = jnp.concatenate([Wg, jnp.zeros((D, D - NC), f32)], axis=1)
    q, dis, st = _gcnprep_call(h, wgp, degcol)
    eacc = _gcn_call(srcp, dstp, eap, q)
    e0 = eacc[:N]
    e1 = eacc[NPAD:NPAD + N]
    bgp = jnp.concatenate([bg, jnp.full((D - NC,), -1e30, f32)])
    out = _final_call(e0, e1, dis, st, bgp.reshape(1, D))
    return out[:, :NC]


# speedup vs baseline: 1.0904x; 1.0013x over previous
import functools

import jax
import jax.numpy as jnp
from jax import lax
from jax.experimental import pallas as pl
from jax.experimental.pallas import tpu as pltpu
from jax.experimental.pallas import tpu_sc as plsc

N = 10000
D = 128
H = 256
NC = 40
EPS = 1e-7

NPAD = 10112
ROWS_PT = NPAD // 16
E_PAD = 323584
CH = 128
IDX_PAD = 12288

@functools.cache
def _mesh():
    return plsc.VectorSubcoreMesh(core_axis_name="c", subcore_axis_name="s",
                                  num_cores=2, num_subcores=16)


def _zero_rows(buf, nrow, ncol):
    z = jnp.zeros((16,), jnp.float32)
    npc = ncol // 16

    def body(i, _):
        buf[i // npc, pl.ds((i % npc) * 16, 16)] = z
        return 0

    lax.fori_loop(0, nrow * npc, body, 0)


def _zero_acc_slice(acc, rows, base, ncol):
    _zero_rows(rows, CH, ncol)
    for j in range(ROWS_PT // CH):
        pltpu.sync_copy(rows, acc.at[pl.ds(base + j * CH, CH)])
    rem = ROWS_PT % CH
    if rem:
        pltpu.sync_copy(rows.at[pl.ds(0, rem)],
                        acc.at[pl.ds(base + (ROWS_PT // CH) * CH, rem)])


def _sc_emb_deg(idx_hbm, emb_hbm, dst_hbm, ea_hbm, h_out, deg_out,
                idxv, didx0, eav0, rows0, didx1, eav1, rows1, acc,
                sem, ssem0, ssem1):
    c = lax.axis_index("c")
    s = lax.axis_index("s")
    w = s * 2 + c
    rpw = IDX_PAD // 32
    for j in range(rpw // CH):
        base = w * rpw + j * CH
        pltpu.sync_copy(idx_hbm.at[pl.ds(base, CH)], idxv)
        pltpu.async_copy(emb_hbm.at[idxv], rows0, sem).wait()
        pltpu.sync_copy(rows0, h_out.at[pl.ds(base, CH)])
    br = s * ROWS_PT
    _zero_acc_slice(acc, rows0, br, D)
    _zero_rows(rows1, CH, D)
    plsc.subcore_barrier()

    epw = E_PAD // 32
    nch = epw // CH
    bufs = ((didx0, eav0, rows0, ssem0), (didx1, eav1, rows1, ssem1))

    def stage(b, chunk):
        didx, eav, rows, _ = bufs[b]
        base = w * epw + chunk * CH
        pltpu.sync_copy(dst_hbm.at[pl.ds(base, CH)], didx)
        pltpu.sync_copy(ea_hbm.at[pl.ds(base, CH)], eav)

        def sb(g, _):
            wv = eav[pl.ds(g * 16, 16)]
            for l in range(16):
                rows[g * 16 + l, pl.ds(0, 16)] = jnp.full(
                    (16,), wv[l], jnp.float32)
            return 0

        lax.fori_loop(0, CH // 16, sb, 0)

    stage(0, 0)

    def body(i, _):
        for b in range(2):
            cur = 2 * i + b
            didx, eav, rows, ssem = bufs[b]
            odidx, _, orows, ossem = bufs[1 - b]
            pltpu.async_copy(rows, acc.at[didx], ssem, add=True)

            @pl.when(cur + 1 < nch)
            def _():
                @pl.when(cur >= 1)
                def _():
                    pltpu.make_async_copy(orows, acc.at[odidx], ossem).wait()

                stage(1 - b, cur + 1)
        return 0

    lax.fori_loop(0, nch // 2, body, 0)
    pltpu.async_copy(rows0, acc.at[didx0], ssem0, add=True)
    pltpu.make_async_copy(rows1, acc.at[didx1], ssem1).wait()
    pltpu.make_async_copy(rows0, acc.at[didx0], ssem0).wait()
    plsc.subcore_barrier()
    pltpu.sync_copy(acc.at[pl.ds(br, ROWS_PT)],
                    deg_out.at[pl.ds(c * NPAD + br, ROWS_PT)])


def _emb_deg_call(idxp, emb, dstp, eap):
    return pl.kernel(
        _sc_emb_deg,
        out_type=[
            jax.ShapeDtypeStruct((IDX_PAD, D), jnp.float32),
            jax.ShapeDtypeStruct((2 * NPAD, D), jnp.float32),
        ],
        mesh=_mesh(),
        scratch_types=[
            pltpu.VMEM((CH,), jnp.int32),
            pltpu.VMEM((CH,), jnp.int32),
            pltpu.VMEM((CH,), jnp.float32),
            pltpu.VMEM((CH, D), jnp.float32),
            pltpu.VMEM((CH,), jnp.int32),
            pltpu.VMEM((CH,), jnp.float32),
            pltpu.VMEM((CH, D), jnp.float32),
            pltpu.VMEM_SHARED((NPAD, D), jnp.float32),
            pltpu.SemaphoreType.DMA,
            pltpu.SemaphoreType.DMA,
            pltpu.SemaphoreType.DMA,
        ],
    )(idxp, emb, dstp, eap)


def _sc_edge(src_hbm, dst_hbm, p_hbm, out_hbm,
             sidx0, didx0, rows0, sidx1, didx1, rows1, acc,
             gsem0, gsem1, ssem0, ssem1):
    c = lax.axis_index("c")
    s = lax.axis_index("s")
    br = s * ROWS_PT
    _zero_acc_slice(acc, rows0, br, D)
    plsc.subcore_barrier()

    coff = c * N
    ept = E_PAD // 16
    nch = ept // CH
    bufs = ((sidx0, didx0, rows0, gsem0, ssem0),
            (sidx1, didx1, rows1, gsem1, ssem1))

    def stage(b, chunk):
        sidx, didx, rows, gsem, _ = bufs[b]
        base = s * ept + chunk * CH
        pltpu.sync_copy(src_hbm.at[pl.ds(base, CH)], sidx)
        pltpu.sync_copy(dst_hbm.at[pl.ds(base, CH)], didx)
        for k in range(CH // 16):
            sidx[pl.ds(k * 16, 16)] = sidx[pl.ds(k * 16, 16)] + coff
        pltpu.async_copy(p_hbm.at[sidx], rows, gsem)

    stage(0, 0)

    def body(i, _):
        for b in range(2):
            cur = 2 * i + b
            sidx, didx, rows, gsem, ssem = bufs[b]
            _, odidx, orows, _, ossem = bufs[1 - b]

            @pl.when(cur + 1 < nch)
            def _():
                @pl.when(cur >= 1)
                def _():
                    pltpu.make_async_copy(orows, acc.at[odidx], ossem).wait()

                stage(1 - b, cur + 1)

            pltpu.make_async_copy(p_hbm.at[sidx], rows, gsem).wait()
            pltpu.async_copy(rows, acc.at[didx], ssem, add=True)
        return 0

    lax.fori_loop(0, nch // 2, body, 0)
    pltpu.make_async_copy(rows0, acc.at[didx0], ssem0).wait()
    pltpu.make_async_copy(rows1, acc.at[didx1], ssem1).wait()
    plsc.subcore_barrier()
    pltpu.sync_copy(acc.at[pl.ds(br, ROWS_PT)],
                    out_hbm.at[pl.ds(c * NPAD + br, ROWS_PT)])


def _edge_call(srcp, dstp, p2):
    return pl.kernel(
        _sc_edge,
        out_type=jax.ShapeDtypeStruct((2 * NPAD, D), jnp.float32),
        mesh=_mesh(),
        scratch_types=(
            [pltpu.VMEM((CH,), jnp.int32),
             pltpu.VMEM((CH,), jnp.int32),
             pltpu.VMEM((CH, D), jnp.float32)] * 2
            + [pltpu.VMEM_SHARED((NPAD, D), jnp.float32)]
            + [pltpu.SemaphoreType.DMA] * 4),
    )(srcp, dstp, p2)


def _sc_gcn(src_hbm, dst_hbm, ea_hbm, q_hbm, out_hbm,
            sidx0, didx0, rows0, sidx1, didx1, rows1,
            sidx2, didx2, rows2, eav, acc,
            gsem0, gsem1, gsem2, ssem0, ssem1, ssem2):
    c = lax.axis_index("c")
    s = lax.axis_index("s")
    w = s * 2 + c
    br = s * ROWS_PT
    _zero_acc_slice(acc, rows0, br, D)
    plsc.subcore_barrier()

    epw = E_PAD // 32
    nch = epw // CH
    bufs = ((sidx0, didx0, rows0, gsem0, ssem0),
            (sidx1, didx1, rows1, gsem1, ssem1),
            (sidx2, didx2, rows2, gsem2, ssem2))

    def stage(b, chunk):
        sidx, didx, rows, gsem, _ = bufs[b]
        base = w * epw + chunk * CH
        pltpu.sync_copy(src_hbm.at[pl.ds(base, CH)], sidx)
        pltpu.sync_copy(dst_hbm.at[pl.ds(base, CH)], didx)
        pltpu.async_copy(q_hbm.at[sidx], rows, gsem)

    def finish(b, chunk):
        sidx, didx, rows, gsem, ssem = bufs[b]
        base = w * epw + chunk * CH
        pltpu.sync_copy(ea_hbm.at[pl.ds(base, CH)], eav)
        pltpu.make_async_copy(q_hbm.at[sidx], rows, gsem).wait()

        def mb(g, _):
            wv = eav[pl.ds(g * 16, 16)]
            for l in range(16):
                wgt = wv[l]
                k = g * 16 + l
                for j in range(3):
                    rows[k, pl.ds(j * 16, 16)] = (
                        rows[k, pl.ds(j * 16, 16)] * wgt)
            return 0

        lax.fori_loop(0, CH // 16, mb, 0)
        pltpu.async_copy(rows, acc.at[didx], ssem, add=True)

    stage(0, 0)
    stage(1, 1)

    def body(i, _):
        for b in range(3):
            cur = 3 * i + b

            @pl.when(cur < nch)
            def _():
                nb = (b + 2) % 3
                _, ndidx, nrows, _, nssem = bufs[nb]

                @pl.when(cur + 2 < nch)
                def _():
                    @pl.when(cur >= 1)
                    def _():
                        pltpu.make_async_copy(
                            nrows, acc.at[ndidx], nssem).wait()

                    stage(nb, cur + 2)

                finish(b, cur)
        return 0

    lax.fori_loop(0, (nch + 2) // 3, body, 0)
    pltpu.make_async_copy(rows0, acc.at[didx0], ssem0).wait()
    pltpu.make_async_copy(rows1, acc.at[didx1], ssem1).wait()
    pltpu.make_async_copy(rows2, acc.at[didx2], ssem2).wait()
    plsc.subcore_barrier()
    pltpu.sync_copy(acc.at[pl.ds(br, ROWS_PT)],
                    out_hbm.at[pl.ds(c * NPAD + br, ROWS_PT)])


def _gcn_call(srcp, dstp, eap, q):
    return pl.kernel(
        _sc_gcn,
        out_type=jax.ShapeDtypeStruct((2 * NPAD, D), jnp.float32),
        mesh=_mesh(),
        scratch_types=(
            [pltpu.VMEM((CH,), jnp.int32),
             pltpu.VMEM((CH,), jnp.int32),
             pltpu.VMEM((CH, D), jnp.float32)] * 3
            + [pltpu.VMEM((CH,), jnp.float32),
               pltpu.VMEM_SHARED((NPAD, D), jnp.float32)]
            + [pltpu.SemaphoreType.DMA] * 6),
    )(srcp, dstp, eap, q)


_BR = 1000


def _ln(v, g, b):
    mu = jnp.mean(v, axis=-1, keepdims=True)
    var = jnp.mean((v - mu) * (v - mu), axis=-1, keepdims=True)
    return (v - mu) * lax.rsqrt(var + 1e-5) * g + b


def _tc_prep(h_ref, g_ref, b_ref, t_ref, r_ref, p_ref):
    h = h_ref[...]
    r = jnp.maximum(_ln(h, g_ref[...], b_ref[...]), 0.0)
    m = r + EPS
    ex = jnp.exp(t_ref[...] * m)
    r_ref[...] = r
    p_ref[0, :, :] = ex
    p_ref[1, :, :] = ex * m


def _prep_call(h, g, b, trow):
    row = lambda i: (i, 0)
    one = lambda i: (0, 0)
    return pl.pallas_call(
        _tc_prep,
        grid=(N // _BR,),
        in_specs=[
            pl.BlockSpec((_BR, D), row),
            pl.BlockSpec((1, D), one),
            pl.BlockSpec((1, D), one),
            pl.BlockSpec((1, D), one),
        ],
        out_specs=[
            pl.BlockSpec((_BR, D), row),
            pl.BlockSpec((2, _BR, D), lambda i: (0, i, 0)),
        ],
        out_shape=[
            jax.ShapeDtypeStruct((N, D), jnp.float32),
            jax.ShapeDtypeStruct((2, N, D), jnp.float32),
        ],
    )(h, g, b, trow)


def _tc_mlp(h_ref, r_ref, den_ref, num_ref, w1_ref, b1_ref, lg_ref, lb_ref,
            w2_ref, b2_ref, h2_ref):
    aggr = num_ref[...] / (den_ref[...] + 1e-16)
    out = aggr + r_ref[...]
    z = jnp.dot(out, w1_ref[...], preferred_element_type=jnp.float32)
    z = _ln(z + b1_ref[...], lg_ref[...], lb_ref[...])
    z = jnp.maximum(z, 0.0)
    z2 = jnp.dot(z, w2_ref[...], preferred_element_type=jnp.float32)
    h2_ref[...] = h_ref[...] + z2 + b2_ref[...]


def _mlp_call(h, r, den, num, w1, b1, lg, lb, w2, b2):
    row = lambda i: (i, 0)
    one = lambda i: (0, 0)
    return pl.pallas_call(
        _tc_mlp,
        grid=(N // _BR,),
        in_specs=[
            pl.BlockSpec((_BR, D), row),
            pl.BlockSpec((_BR, D), row),
            pl.BlockSpec((_BR, D), row),
            pl.BlockSpec((_BR, D), row),
            pl.BlockSpec((D, H), one),
            pl.BlockSpec((1, H), one),
            pl.BlockSpec((1, H), one),
            pl.BlockSpec((1, H), one),
            pl.BlockSpec((H, D), one),
            pl.BlockSpec((1, D), one),
        ],
        out_specs=pl.BlockSpec((_BR, D), row),
        out_shape=jax.ShapeDtypeStruct((N, D), jnp.float32),
    )(h, r, den, num, w1, b1, lg, lb, w2, b2)


def _tc_degsum(d_ref, o_ref):
    d = d_ref[...]
    o_ref[...] = d[0:NPAD, 0:1] + d[NPAD:2 * NPAD, 0:1] + 1.0


def _degsum_call(deg2):
    return pl.pallas_call(
        _tc_degsum,
        out_shape=jax.ShapeDtypeStruct((NPAD, 1), jnp.float32),
    )(deg2)


def _tc_gcnprep(h_ref, wg_ref, deg_ref, q_ref, dis_ref, st_ref):
    hw = jnp.dot(h_ref[...], wg_ref[...], preferred_element_type=jnp.float32)
    deg = deg_ref[...]
    dis = jnp.where(deg > 0, 1.0 / jnp.sqrt(deg), 0.0)
    q_ref[...] = hw * dis
    dis_ref[...] = dis
    st_ref[...] = hw * (dis * dis)


def _gcnprep_call(h, wgp, degcol):
    row = lambda i: (i, 0)
    one = lambda i: (0, 0)
    return pl.pallas_call(
        _tc_gcnprep,
        grid=(N // _BR,),
        in_specs=[
            pl.BlockSpec((_BR, D), row),
            pl.BlockSpec((D, D), one),
            pl.BlockSpec((_BR, 1), row),
        ],
        out_specs=[
            pl.BlockSpec((_BR, D), row),
            pl.BlockSpec((_BR, 1), row),
            pl.BlockSpec((_BR, D), row),
        ],
        out_shape=[
            jax.ShapeDtypeStruct((N, D), jnp.float32),
            jax.ShapeDtypeStruct((N, 1), jnp.float32),
            jax.ShapeDtypeStruct((N, D), jnp.float32),
        ],
    )(h, wgp, degcol)


def _tc_final(e0_ref, e1_ref, dis_ref, st_ref, bg_ref, o_ref):
    o = dis_ref[...] * (e0_ref[...] + e1_ref[...]) + st_ref[...] + bg_ref[...]
    mx = jnp.max(o, axis=-1, keepdims=True)
    lse = jnp.log(jnp.sum(jnp.exp(o - mx), axis=-1, keepdims=True))
    o_ref[...] = o - mx - lse


def _final_call(e0, e1, dis, st, bgp):
    row = lambda i: (i, 0)
    one = lambda i: (0, 0)
    return pl.pallas_call(
        _tc_final,
        grid=(N // _BR,),
        in_specs=[
            pl.BlockSpec((_BR, D), row),
            pl.BlockSpec((_BR, D), row),
            pl.BlockSpec((_BR, 1), row),
            pl.BlockSpec((_BR, D), row),
            pl.BlockSpec((1, D), one),
        ],
        out_specs=pl.BlockSpec((_BR, D), row),
        out_shape=jax.ShapeDtypeStruct((N, D), jnp.float32),
    )(e0, e1, dis, st, bgp)


def kernel(x, edge_index, edge_attr, emb, ln_g, ln_b, t, W1, b1,
           mlp_ln_g, mlp_ln_b, W2, b2, Wg, bg):
    f32 = jnp.float32
    src = edge_index[0]
    dst = edge_index[1]
    e = src.shape[0]
    npad_e = E_PAD - e
    srcp = jnp.concatenate([src, jnp.zeros((npad_e,), src.dtype)])
    dstp = jnp.concatenate([dst, jnp.full((npad_e,), N, dst.dtype)])
    eap = jnp.concatenate([edge_attr, jnp.zeros((npad_e,), f32)])
    idxp = jnp.concatenate(
        [x[:, 0].astype(jnp.int32),
         jnp.zeros((IDX_PAD - x.shape[0],), jnp.int32)])

    hfull, deg2 = _emb_deg_call(idxp, emb, dstp, eap)
    h = hfull[:N]
    degcol = _degsum_call(deg2)[:N]

    for i in range(2):
        trow = jnp.full((1, D), t[i], f32)
        r, p2 = _prep_call(h, ln_g[i].reshape(1, D), ln_b[i].reshape(1, D),
                           trow)
        s2 = _edge_call(srcp, dstp, p2.reshape(2 * N, D))
        den = s2[:N]
        num = s2[NPAD:NPAD + N]
        h = _mlp_call(h, r, den, num, W1[i], b1[i].reshape(1, H),
                      mlp_ln_g[i].reshape(1, H), mlp_ln_b[i].reshape(1, H),
                      W2[i], b2[i].reshape(1, D))

    wgp = jnp.concatenate([Wg, jnp.zeros((D, D - NC), f32)], axis=1)
    q, dis, st = _gcnprep_call(h, wgp, degcol)
    eacc = _gcn_call(srcp, dstp, eap, q)
    e0 = eacc[:N]
    e1 = eacc[NPAD:NPAD + N]
    bgp = jnp.concatenate([bg, jnp.full((D - NC,), -1e30, f32)])
    out = _final_call(e0, e1, dis, st, bgp.reshape(1, D))
    return out[:, :NC]


# speedup vs baseline: 1.1368x; 1.0426x over previous
import functools

import jax
import jax.numpy as jnp
from jax import lax
from jax.experimental import pallas as pl
from jax.experimental.pallas import tpu as pltpu
from jax.experimental.pallas import tpu_sc as plsc

N = 10000
D = 128
H = 256
NC = 40
EPS = 1e-7

NPAD = 10112
ROWS_PT = NPAD // 16
E_PAD = 323584
CH = 128
IDX_PAD = 12288

@functools.cache
def _mesh():
    return plsc.VectorSubcoreMesh(core_axis_name="c", subcore_axis_name="s",
                                  num_cores=2, num_subcores=16)


def _zero_rows(buf, nrow, ncol):
    z = jnp.zeros((16,), jnp.float32)
    npc = ncol // 16

    def body(i, _):
        buf[i // npc, pl.ds((i % npc) * 16, 16)] = z
        return 0

    lax.fori_loop(0, nrow * npc, body, 0)


def _zero_acc_slice(acc, rows, base, ncol):
    _zero_rows(rows, CH, ncol)
    for j in range(ROWS_PT // CH):
        pltpu.sync_copy(rows, acc.at[pl.ds(base + j * CH, CH)])
    rem = ROWS_PT % CH
    if rem:
        pltpu.sync_copy(rows.at[pl.ds(0, rem)],
                        acc.at[pl.ds(base + (ROWS_PT // CH) * CH, rem)])


def _sc_emb(idx_hbm, emb_hbm, h_out, idxv, rows0, sem):
    c = lax.axis_index("c")
    s = lax.axis_index("s")
    w = s * 2 + c
    rpw = IDX_PAD // 32
    for j in range(rpw // CH):
        base = w * rpw + j * CH
        pltpu.sync_copy(idx_hbm.at[pl.ds(base, CH)], idxv)
        pltpu.async_copy(emb_hbm.at[idxv], rows0, sem).wait()
        pltpu.sync_copy(rows0, h_out.at[pl.ds(base, CH)])


def _emb_call(idxp, emb):
    return pl.kernel(
        _sc_emb,
        out_type=jax.ShapeDtypeStruct((IDX_PAD, D), jnp.float32),
        mesh=_mesh(),
        scratch_types=[
            pltpu.VMEM((CH,), jnp.int32),
            pltpu.VMEM((CH, D), jnp.float32),
            pltpu.SemaphoreType.DMA,
        ],
    )(idxp, emb)


def _sc_deg(dst_hbm, ea_hbm, deg_out,
            didx0, eav0, rows0, didx1, eav1, rows1, acc,
            ssem0, ssem1):
    c = lax.axis_index("c")
    s = lax.axis_index("s")
    w = s * 2 + c
    br = s * ROWS_PT
    _zero_acc_slice(acc, rows0, br, D)
    _zero_rows(rows1, CH, D)
    plsc.subcore_barrier()

    epw = E_PAD // 32
    nch = epw // CH
    bufs = ((didx0, eav0, rows0, ssem0), (didx1, eav1, rows1, ssem1))

    def stage(b, chunk):
        didx, eav, rows, _ = bufs[b]
        base = w * epw + chunk * CH
        pltpu.sync_copy(dst_hbm.at[pl.ds(base, CH)], didx)
        pltpu.sync_copy(ea_hbm.at[pl.ds(base, CH)], eav)

        def sb(g, _):
            wv = eav[pl.ds(g * 16, 16)]
            for l in range(16):
                rows[g * 16 + l, pl.ds(0, 16)] = jnp.full(
                    (16,), wv[l], jnp.float32)
            return 0

        lax.fori_loop(0, CH // 16, sb, 0)

    stage(0, 0)

    def body(i, _):
        for b in range(2):
            cur = 2 * i + b
            didx, eav, rows, ssem = bufs[b]
            odidx, _, orows, ossem = bufs[1 - b]
            pltpu.async_copy(rows, acc.at[didx], ssem, add=True)

            @pl.when(cur + 1 < nch)
            def _():
                @pl.when(cur >= 1)
                def _():
                    pltpu.make_async_copy(orows, acc.at[odidx], ossem).wait()

                stage(1 - b, cur + 1)
        return 0

    lax.fori_loop(0, nch // 2, body, 0)
    pltpu.async_copy(rows0, acc.at[didx0], ssem0, add=True)
    pltpu.make_async_copy(rows1, acc.at[didx1], ssem1).wait()
    pltpu.make_async_copy(rows0, acc.at[didx0], ssem0).wait()
    plsc.subcore_barrier()
    pltpu.sync_copy(acc.at[pl.ds(br, ROWS_PT)],
                    deg_out.at[pl.ds(c * NPAD + br, ROWS_PT)])


def _deg_call(dstp, eap):
    return pl.kernel(
        _sc_deg,
        out_type=jax.ShapeDtypeStruct((2 * NPAD, D), jnp.float32),
        mesh=_mesh(),
        scratch_types=[
            pltpu.VMEM((CH,), jnp.int32),
            pltpu.VMEM((CH,), jnp.float32),
            pltpu.VMEM((CH, D), jnp.float32),
            pltpu.VMEM((CH,), jnp.int32),
            pltpu.VMEM((CH,), jnp.float32),
            pltpu.VMEM((CH, D), jnp.float32),
            pltpu.VMEM_SHARED((NPAD, D), jnp.float32),
            pltpu.SemaphoreType.DMA,
            pltpu.SemaphoreType.DMA,
        ],
    )(dstp, eap)


def _sc_edge(src_hbm, dst_hbm, p_hbm, out_hbm,
             sidx0, didx0, rows0, sidx1, didx1, rows1, acc,
             gsem0, gsem1, ssem0, ssem1):
    c = lax.axis_index("c")
    s = lax.axis_index("s")
    br = s * ROWS_PT
    _zero_acc_slice(acc, rows0, br, D)
    plsc.subcore_barrier()

    coff = c * N
    ept = E_PAD // 16
    nch = ept // CH
    bufs = ((sidx0, didx0, rows0, gsem0, ssem0),
            (sidx1, didx1, rows1, gsem1, ssem1))

    def stage(b, chunk):
        sidx, didx, rows, gsem, _ = bufs[b]
        base = s * ept + chunk * CH
        pltpu.sync_copy(src_hbm.at[pl.ds(base, CH)], sidx)
        pltpu.sync_copy(dst_hbm.at[pl.ds(base, CH)], didx)
        for k in range(CH // 16):
            sidx[pl.ds(k * 16, 16)] = sidx[pl.ds(k * 16, 16)] + coff
        pltpu.async_copy(p_hbm.at[sidx], rows, gsem)

    stage(0, 0)

    def body(i, _):
        for b in range(2):
            cur = 2 * i + b
            sidx, didx, rows, gsem, ssem = bufs[b]
            _, odidx, orows, _, ossem = bufs[1 - b]

            @pl.when(cur + 1 < nch)
            def _():
                @pl.when(cur >= 1)
                def _():
                    pltpu.make_async_copy(orows, acc.at[odidx], ossem).wait()

                stage(1 - b, cur + 1)

            pltpu.make_async_copy(p_hbm.at[sidx], rows, gsem).wait()
            pltpu.async_copy(rows, acc.at[didx], ssem, add=True)
        return 0

    lax.fori_loop(0, nch // 2, body, 0)
    pltpu.make_async_copy(rows0, acc.at[didx0], ssem0).wait()
    pltpu.make_async_copy(rows1, acc.at[didx1], ssem1).wait()
    plsc.subcore_barrier()
    pltpu.sync_copy(acc.at[pl.ds(br, ROWS_PT)],
                    out_hbm.at[pl.ds(c * NPAD + br, ROWS_PT)])


def _edge_call(srcp, dstp, p2):
    return pl.kernel(
        _sc_edge,
        out_type=jax.ShapeDtypeStruct((2 * NPAD, D), jnp.float32),
        mesh=_mesh(),
        scratch_types=(
            [pltpu.VMEM((CH,), jnp.int32),
             pltpu.VMEM((CH,), jnp.int32),
             pltpu.VMEM((CH, D), jnp.float32)] * 2
            + [pltpu.VMEM_SHARED((NPAD, D), jnp.float32)]
            + [pltpu.SemaphoreType.DMA] * 4),
    )(srcp, dstp, p2)


def _sc_gcn(src_hbm, dst_hbm, ea_hbm, q_hbm, out_hbm,
            sidx0, didx0, rows0, sidx1, didx1, rows1,
            sidx2, didx2, rows2, eav, acc,
            gsem0, gsem1, gsem2, ssem0, ssem1, ssem2):
    c = lax.axis_index("c")
    s = lax.axis_index("s")
    w = s * 2 + c
    br = s * ROWS_PT
    _zero_acc_slice(acc, rows0, br, D)
    plsc.subcore_barrier()

    epw = E_PAD // 32
    nch = epw // CH
    bufs = ((sidx0, didx0, rows0, gsem0, ssem0),
            (sidx1, didx1, rows1, gsem1, ssem1),
            (sidx2, didx2, rows2, gsem2, ssem2))

    def stage(b, chunk):
        sidx, didx, rows, gsem, _ = bufs[b]
        base = w * epw + chunk * CH
        pltpu.sync_copy(src_hbm.at[pl.ds(base, CH)], sidx)
        pltpu.sync_copy(dst_hbm.at[pl.ds(base, CH)], didx)
        pltpu.async_copy(q_hbm.at[sidx], rows, gsem)

    def finish(b, chunk):
        sidx, didx, rows, gsem, ssem = bufs[b]
        base = w * epw + chunk * CH
        pltpu.sync_copy(ea_hbm.at[pl.ds(base, CH)], eav)
        pltpu.make_async_copy(q_hbm.at[sidx], rows, gsem).wait()

        def mb(g, _):
            wv = eav[pl.ds(g * 16, 16)]
            for l in range(16):
                wgt = wv[l]
                k = g * 16 + l
                for j in range(3):
                    rows[k, pl.ds(j * 16, 16)] = (
                        rows[k, pl.ds(j * 16, 16)] * wgt)
            return 0

        lax.fori_loop(0, CH // 16, mb, 0)
        pltpu.async_copy(rows, acc.at[didx], ssem, add=True)

    stage(0, 0)
    stage(1, 1)

    def body(i, _):
        for b in range(3):
            cur = 3 * i + b

            @pl.when(cur < nch)
            def _():
                nb = (b + 2) % 3
                _, ndidx, nrows, _, nssem = bufs[nb]

                @pl.when(cur + 2 < nch)
                def _():
                    @pl.when(cur >= 1)
                    def _():
                        pltpu.make_async_copy(
                            nrows, acc.at[ndidx], nssem).wait()

                    stage(nb, cur + 2)

                finish(b, cur)
        return 0

    lax.fori_loop(0, (nch + 2) // 3, body, 0)
    pltpu.make_async_copy(rows0, acc.at[didx0], ssem0).wait()
    pltpu.make_async_copy(rows1, acc.at[didx1], ssem1).wait()
    pltpu.make_async_copy(rows2, acc.at[didx2], ssem2).wait()
    plsc.subcore_barrier()
    pltpu.sync_copy(acc.at[pl.ds(br, ROWS_PT)],
                    out_hbm.at[pl.ds(c * NPAD + br, ROWS_PT)])


def _gcn_call(srcp, dstp, eap, q):
    return pl.kernel(
        _sc_gcn,
        out_type=jax.ShapeDtypeStruct((2 * NPAD, D), jnp.float32),
        mesh=_mesh(),
        scratch_types=(
            [pltpu.VMEM((CH,), jnp.int32),
             pltpu.VMEM((CH,), jnp.int32),
             pltpu.VMEM((CH, D), jnp.float32)] * 3
            + [pltpu.VMEM((CH,), jnp.float32),
               pltpu.VMEM_SHARED((NPAD, D), jnp.float32)]
            + [pltpu.SemaphoreType.DMA] * 6),
    )(srcp, dstp, eap, q)


_BR = 1000


def _ln(v, g, b):
    mu = jnp.mean(v, axis=-1, keepdims=True)
    var = jnp.mean((v - mu) * (v - mu), axis=-1, keepdims=True)
    return (v - mu) * lax.rsqrt(var + 1e-5) * g + b


def _tc_prep(h_ref, g_ref, b_ref, t_ref, r_ref, p_ref):
    h = h_ref[...]
    r = jnp.maximum(_ln(h, g_ref[...], b_ref[...]), 0.0)
    m = r + EPS
    ex = jnp.exp(t_ref[...] * m)
    r_ref[...] = r
    p_ref[0, :, :] = ex
    p_ref[1, :, :] = ex * m


def _prep_call(h, g, b, trow):
    row = lambda i: (i, 0)
    one = lambda i: (0, 0)
    return pl.pallas_call(
        _tc_prep,
        grid=(N // _BR,),
        in_specs=[
            pl.BlockSpec((_BR, D), row),
            pl.BlockSpec((1, D), one),
            pl.BlockSpec((1, D), one),
            pl.BlockSpec((1, D), one),
        ],
        out_specs=[
            pl.BlockSpec((_BR, D), row),
            pl.BlockSpec((2, _BR, D), lambda i: (0, i, 0)),
        ],
        out_shape=[
            jax.ShapeDtypeStruct((N, D), jnp.float32),
            jax.ShapeDtypeStruct((2, N, D), jnp.float32),
        ],
    )(h, g, b, trow)


def _tc_mlp(h_ref, r_ref, den_ref, num_ref, w1_ref, b1_ref, lg_ref, lb_ref,
            w2_ref, b2_ref, h2_ref):
    aggr = num_ref[...] / (den_ref[...] + 1e-16)
    out = aggr + r_ref[...]
    z = jnp.dot(out, w1_ref[...], preferred_element_type=jnp.float32)
    z = _ln(z + b1_ref[...], lg_ref[...], lb_ref[...])
    z = jnp.maximum(z, 0.0)
    z2 = jnp.dot(z, w2_ref[...], preferred_element_type=jnp.float32)
    h2_ref[...] = h_ref[...] + z2 + b2_ref[...]


def _mlp_call(h, r, den, num, w1, b1, lg, lb, w2, b2):
    row = lambda i: (i, 0)
    one = lambda i: (0, 0)
    return pl.pallas_call(
        _tc_mlp,
        grid=(N // _BR,),
        in_specs=[
            pl.BlockSpec((_BR, D), row),
            pl.BlockSpec((_BR, D), row),
            pl.BlockSpec((_BR, D), row),
            pl.BlockSpec((_BR, D), row),
            pl.BlockSpec((D, H), one),
            pl.BlockSpec((1, H), one),
            pl.BlockSpec((1, H), one),
            pl.BlockSpec((1, H), one),
            pl.BlockSpec((H, D), one),
            pl.BlockSpec((1, D), one),
        ],
        out_specs=pl.BlockSpec((_BR, D), row),
        out_shape=jax.ShapeDtypeStruct((N, D), jnp.float32),
    )(h, r, den, num, w1, b1, lg, lb, w2, b2)


def _tc_degsum(d_ref, o_ref):
    d = d_ref[...]
    o_ref[...] = d[0:NPAD, 0:1] + d[NPAD:2 * NPAD, 0:1] + 1.0


def _degsum_call(deg2):
    return pl.pallas_call(
        _tc_degsum,
        out_shape=jax.ShapeDtypeStruct((NPAD, 1), jnp.float32),
    )(deg2)


def _tc_gcnprep(h_ref, wg_ref, deg_ref, q_ref, dis_ref, st_ref):
    hw = jnp.dot(h_ref[...], wg_ref[...], preferred_element_type=jnp.float32)
    deg = deg_ref[...]
    dis = jnp.where(deg > 0, 1.0 / jnp.sqrt(deg), 0.0)
    q_ref[...] = hw * dis
    dis_ref[...] = dis
    st_ref[...] = hw * (dis * dis)


def _gcnprep_call(h, wgp, degcol):
    row = lambda i: (i, 0)
    one = lambda i: (0, 0)
    return pl.pallas_call(
        _tc_gcnprep,
        grid=(N // _BR,),
        in_specs=[
            pl.BlockSpec((_BR, D), row),
            pl.BlockSpec((D, D), one),
            pl.BlockSpec((_BR, 1), row),
        ],
        out_specs=[
            pl.BlockSpec((_BR, D), row),
            pl.BlockSpec((_BR, 1), row),
            pl.BlockSpec((_BR, D), row),
        ],
        out_shape=[
            jax.ShapeDtypeStruct((N, D), jnp.float32),
            jax.ShapeDtypeStruct((N, 1), jnp.float32),
            jax.ShapeDtypeStruct((N, D), jnp.float32),
        ],
    )(h, wgp, degcol)


def _tc_final(e0_ref, e1_ref, dis_ref, st_ref, bg_ref, o_ref):
    o = dis_ref[...] * (e0_ref[...] + e1_ref[...]) + st_ref[...] + bg_ref[...]
    mx = jnp.max(o, axis=-1, keepdims=True)
    lse = jnp.log(jnp.sum(jnp.exp(o - mx), axis=-1, keepdims=True))
    o_ref[...] = o - mx - lse


def _final_call(e0, e1, dis, st, bgp):
    row = lambda i: (i, 0)
    one = lambda i: (0, 0)
    return pl.pallas_call(
        _tc_final,
        grid=(N // _BR,),
        in_specs=[
            pl.BlockSpec((_BR, D), row),
            pl.BlockSpec((_BR, D), row),
            pl.BlockSpec((_BR, 1), row),
            pl.BlockSpec((_BR, D), row),
            pl.BlockSpec((1, D), one),
        ],
        out_specs=pl.BlockSpec((_BR, D), row),
        out_shape=jax.ShapeDtypeStruct((N, D), jnp.float32),
    )(e0, e1, dis, st, bgp)


def kernel(x, edge_index, edge_attr, emb, ln_g, ln_b, t, W1, b1,
           mlp_ln_g, mlp_ln_b, W2, b2, Wg, bg):
    f32 = jnp.float32
    src = edge_index[0]
    dst = edge_index[1]
    e = src.shape[0]
    npad_e = E_PAD - e
    srcp = jnp.concatenate([src, jnp.zeros((npad_e,), src.dtype)])
    dstp = jnp.concatenate([dst, jnp.full((npad_e,), N, dst.dtype)])
    eap = jnp.concatenate([edge_attr, jnp.zeros((npad_e,), f32)])
    idxp = jnp.concatenate(
        [x[:, 0].astype(jnp.int32),
         jnp.zeros((IDX_PAD - x.shape[0],), jnp.int32)])

    hfull = _emb_call(idxp, emb)
    h = hfull[:N]
    deg2 = _deg_call(dstp, eap)
    degcol = _degsum_call(deg2)[:N]

    for i in range(2):
        trow = jnp.full((1, D), t[i], f32)
        r, p2 = _prep_call(h, ln_g[i].reshape(1, D), ln_b[i].reshape(1, D),
                           trow)
        s2 = _edge_call(srcp, dstp, p2.reshape(2 * N, D))
        den = s2[:N]
        num = s2[NPAD:NPAD + N]
        h = _mlp_call(h, r, den, num, W1[i], b1[i].reshape(1, H),
                      mlp_ln_g[i].reshape(1, H), mlp_ln_b[i].reshape(1, H),
                      W2[i], b2[i].reshape(1, D))

    wgp = jnp.concatenate([Wg, jnp.zeros((D, D - NC), f32)], axis=1)
    q, dis, st = _gcnprep_call(h, wgp, degcol)
    eacc = _gcn_call(srcp, dstp, eap, q)
    e0 = eacc[:N]
    e1 = eacc[NPAD:NPAD + N]
    bgp = jnp.concatenate([bg, jnp.full((D - NC,), -1e30, f32)])
    out = _final_call(e0, e1, dis, st, bgp.reshape(1, D))
    return out[:, :NC]


# speedup vs baseline: 1.1375x; 1.0006x over previous
import functools

import jax
import jax.numpy as jnp
from jax import lax
from jax.experimental import pallas as pl
from jax.experimental.pallas import tpu as pltpu
from jax.experimental.pallas import tpu_sc as plsc

N = 10000
D = 128
H = 256
NC = 40
EPS = 1e-7

NPAD = 10112
ROWS_PT = NPAD // 16
E_PAD = 323584
CH = 128
IDX_PAD = 12288

@functools.cache
def _mesh():
    return plsc.VectorSubcoreMesh(core_axis_name="c", subcore_axis_name="s",
                                  num_cores=2, num_subcores=16)


def _zero_rows(buf, nrow, ncol):
    z = jnp.zeros((16,), jnp.float32)
    npc = ncol // 16

    def body(i, _):
        buf[i // npc, pl.ds((i % npc) * 16, 16)] = z
        return 0

    lax.fori_loop(0, nrow * npc, body, 0)


def _zero_acc_slice(acc, rows, base, ncol):
    _zero_rows(rows, CH, ncol)
    for j in range(ROWS_PT // CH):
        pltpu.sync_copy(rows, acc.at[pl.ds(base + j * CH, CH)])
    rem = ROWS_PT % CH
    if rem:
        pltpu.sync_copy(rows.at[pl.ds(0, rem)],
                        acc.at[pl.ds(base + (ROWS_PT // CH) * CH, rem)])


def _sc_emb(idx_hbm, emb_hbm, h_out,
            idxv0, rows0, idxv1, rows1, idxv2, rows2,
            gsem0, gsem1, gsem2, osem0, osem1, osem2):
    c = lax.axis_index("c")
    s = lax.axis_index("s")
    w = s * 2 + c
    rpw = IDX_PAD // 32
    bufs = ((idxv0, rows0, gsem0, osem0),
            (idxv1, rows1, gsem1, osem1),
            (idxv2, rows2, gsem2, osem2))
    nch = rpw // CH
    for j in range(nch):
        idxv, rows, gsem, _ = bufs[j]
        pltpu.sync_copy(idx_hbm.at[pl.ds(w * rpw + j * CH, CH)], idxv)
        pltpu.async_copy(emb_hbm.at[idxv], rows, gsem)
    for j in range(nch):
        idxv, rows, gsem, osem = bufs[j]
        pltpu.make_async_copy(emb_hbm.at[idxv], rows, gsem).wait()
        pltpu.async_copy(rows, h_out.at[pl.ds(w * rpw + j * CH, CH)], osem)
    for j in range(nch):
        _, rows, _, osem = bufs[j]
        pltpu.make_async_copy(
            rows, h_out.at[pl.ds(w * rpw + j * CH, CH)], osem).wait()


def _emb_call(idxp, emb):
    return pl.kernel(
        _sc_emb,
        out_type=jax.ShapeDtypeStruct((IDX_PAD, D), jnp.float32),
        mesh=_mesh(),
        scratch_types=(
            [pltpu.VMEM((CH,), jnp.int32),
             pltpu.VMEM((CH, D), jnp.float32)] * 3
            + [pltpu.SemaphoreType.DMA] * 6),
    )(idxp, emb)


def _sc_deg(dst_hbm, ea_hbm, deg_out,
            didx0, eav0, rows0, didx1, eav1, rows1, acc,
            ssem0, ssem1):
    c = lax.axis_index("c")
    s = lax.axis_index("s")
    w = s * 2 + c
    br = s * ROWS_PT
    _zero_acc_slice(acc, rows0, br, D)
    _zero_rows(rows1, CH, D)
    plsc.subcore_barrier()

    epw = E_PAD // 32
    nch = epw // CH
    bufs = ((didx0, eav0, rows0, ssem0), (didx1, eav1, rows1, ssem1))

    def stage(b, chunk):
        didx, eav, rows, _ = bufs[b]
        base = w * epw + chunk * CH
        pltpu.sync_copy(dst_hbm.at[pl.ds(base, CH)], didx)
        pltpu.sync_copy(ea_hbm.at[pl.ds(base, CH)], eav)

        def sb(g, _):
            wv = eav[pl.ds(g * 16, 16)]
            for l in range(16):
                rows[g * 16 + l, pl.ds(0, 16)] = jnp.full(
                    (16,), wv[l], jnp.float32)
            return 0

        lax.fori_loop(0, CH // 16, sb, 0)

    stage(0, 0)

    def body(i, _):
        for b in range(2):
            cur = 2 * i + b
            didx, eav, rows, ssem = bufs[b]
            odidx, _, orows, ossem = bufs[1 - b]
            pltpu.async_copy(rows, acc.at[didx], ssem, add=True)

            @pl.when(cur + 1 < nch)
            def _():
                @pl.when(cur >= 1)
                def _():
                    pltpu.make_async_copy(orows, acc.at[odidx], ossem).wait()

                stage(1 - b, cur + 1)
        return 0

    lax.fori_loop(0, nch // 2, body, 0)
    pltpu.async_copy(rows0, acc.at[didx0], ssem0, add=True)
    pltpu.make_async_copy(rows1, acc.at[didx1], ssem1).wait()
    pltpu.make_async_copy(rows0, acc.at[didx0], ssem0).wait()
    plsc.subcore_barrier()
    pltpu.sync_copy(acc.at[pl.ds(br, ROWS_PT)],
                    deg_out.at[pl.ds(c * NPAD + br, ROWS_PT)])


def _deg_call(dstp, eap):
    return pl.kernel(
        _sc_deg,
        out_type=jax.ShapeDtypeStruct((2 * NPAD, D), jnp.float32),
        mesh=_mesh(),
        scratch_types=[
            pltpu.VMEM((CH,), jnp.int32),
            pltpu.VMEM((CH,), jnp.float32),
            pltpu.VMEM((CH, D), jnp.float32),
            pltpu.VMEM((CH,), jnp.int32),
            pltpu.VMEM((CH,), jnp.float32),
            pltpu.VMEM((CH, D), jnp.float32),
            pltpu.VMEM_SHARED((NPAD, D), jnp.float32),
            pltpu.SemaphoreType.DMA,
            pltpu.SemaphoreType.DMA,
        ],
    )(dstp, eap)


def _sc_edge(src_hbm, dst_hbm, p_hbm, out_hbm,
             sidx0, didx0, rows0, sidx1, didx1, rows1, acc,
             gsem0, gsem1, ssem0, ssem1):
    c = lax.axis_index("c")
    s = lax.axis_index("s")
    br = s * ROWS_PT
    _zero_acc_slice(acc, rows0, br, D)
    plsc.subcore_barrier()

    coff = c * N
    ept = E_PAD // 16
    nch = ept // CH
    bufs = ((sidx0, didx0, rows0, gsem0, ssem0),
            (sidx1, didx1, rows1, gsem1, ssem1))

    def stage(b, chunk):
        sidx, didx, rows, gsem, _ = bufs[b]
        base = s * ept + chunk * CH
        pltpu.sync_copy(src_hbm.at[pl.ds(base, CH)], sidx)
        pltpu.sync_copy(dst_hbm.at[pl.ds(base, CH)], didx)
        for k in range(CH // 16):
            sidx[pl.ds(k * 16, 16)] = sidx[pl.ds(k * 16, 16)] + coff
        pltpu.async_copy(p_hbm.at[sidx], rows, gsem)

    stage(0, 0)

    def body(i, _):
        for b in range(2):
            cur = 2 * i + b
            sidx, didx, rows, gsem, ssem = bufs[b]
            _, odidx, orows, _, ossem = bufs[1 - b]

            @pl.when(cur + 1 < nch)
            def _():
                @pl.when(cur >= 1)
                def _():
                    pltpu.make_async_copy(orows, acc.at[odidx], ossem).wait()

                stage(1 - b, cur + 1)

            pltpu.make_async_copy(p_hbm.at[sidx], rows, gsem).wait()
            pltpu.async_copy(rows, acc.at[didx], ssem, add=True)
        return 0

    lax.fori_loop(0, nch // 2, body, 0)
    pltpu.make_async_copy(rows0, acc.at[didx0], ssem0).wait()
    pltpu.make_async_copy(rows1, acc.at[didx1], ssem1).wait()
    plsc.subcore_barrier()
    pltpu.sync_copy(acc.at[pl.ds(br, ROWS_PT)],
                    out_hbm.at[pl.ds(c * NPAD + br, ROWS_PT)])


def _edge_call(srcp, dstp, p2):
    return pl.kernel(
        _sc_edge,
        out_type=jax.ShapeDtypeStruct((2 * NPAD, D), jnp.float32),
        mesh=_mesh(),
        scratch_types=(
            [pltpu.VMEM((CH,), jnp.int32),
             pltpu.VMEM((CH,), jnp.int32),
             pltpu.VMEM((CH, D), jnp.float32)] * 2
            + [pltpu.VMEM_SHARED((NPAD, D), jnp.float32)]
            + [pltpu.SemaphoreType.DMA] * 4),
    )(srcp, dstp, p2)


def _sc_gcn(src_hbm, dst_hbm, ea_hbm, q_hbm, out_hbm,
            sidx0, didx0, rows0, sidx1, didx1, rows1,
            sidx2, didx2, rows2, eav, acc,
            gsem0, gsem1, gsem2, ssem0, ssem1, ssem2):
    c = lax.axis_index("c")
    s = lax.axis_index("s")
    w = s * 2 + c
    br = s * ROWS_PT
    _zero_acc_slice(acc, rows0, br, D)
    plsc.subcore_barrier()

    epw = E_PAD // 32
    nch = epw // CH
    bufs = ((sidx0, didx0, rows0, gsem0, ssem0),
            (sidx1, didx1, rows1, gsem1, ssem1),
            (sidx2, didx2, rows2, gsem2, ssem2))

    def stage(b, chunk):
        sidx, didx, rows, gsem, _ = bufs[b]
        base = w * epw + chunk * CH
        pltpu.sync_copy(src_hbm.at[pl.ds(base, CH)], sidx)
        pltpu.sync_copy(dst_hbm.at[pl.ds(base, CH)], didx)
        pltpu.async_copy(q_hbm.at[sidx], rows, gsem)

    def finish(b, chunk):
        sidx, didx, rows, gsem, ssem = bufs[b]
        base = w * epw + chunk * CH
        pltpu.sync_copy(ea_hbm.at[pl.ds(base, CH)], eav)
        pltpu.make_async_copy(q_hbm.at[sidx], rows, gsem).wait()

        def mb(g, _):
            wv = eav[pl.ds(g * 16, 16)]
            for l in range(16):
                wgt = wv[l]
                k = g * 16 + l
                for j in range(3):
                    rows[k, pl.ds(j * 16, 16)] = (
                        rows[k, pl.ds(j * 16, 16)] * wgt)
            return 0

        lax.fori_loop(0, CH // 16, mb, 0)
        pltpu.async_copy(rows, acc.at[didx], ssem, add=True)

    stage(0, 0)
    stage(1, 1)

    def body(i, _):
        for b in range(3):
            cur = 3 * i + b

            @pl.when(cur < nch)
            def _():
                nb = (b + 2) % 3
                _, ndidx, nrows, _, nssem = bufs[nb]

                @pl.when(cur + 2 < nch)
                def _():
                    @pl.when(cur >= 1)
                    def _():
                        pltpu.make_async_copy(
                            nrows, acc.at[ndidx], nssem).wait()

                    stage(nb, cur + 2)

                finish(b, cur)
        return 0

    lax.fori_loop(0, (nch + 2) // 3, body, 0)
    pltpu.make_async_copy(rows0, acc.at[didx0], ssem0).wait()
    pltpu.make_async_copy(rows1, acc.at[didx1], ssem1).wait()
    pltpu.make_async_copy(rows2, acc.at[didx2], ssem2).wait()
    plsc.subcore_barrier()
    pltpu.sync_copy(acc.at[pl.ds(br, ROWS_PT)],
                    out_hbm.at[pl.ds(c * NPAD + br, ROWS_PT)])


def _gcn_call(srcp, dstp, eap, q):
    return pl.kernel(
        _sc_gcn,
        out_type=jax.ShapeDtypeStruct((2 * NPAD, D), jnp.float32),
        mesh=_mesh(),
        scratch_types=(
            [pltpu.VMEM((CH,), jnp.int32),
             pltpu.VMEM((CH,), jnp.int32),
             pltpu.VMEM((CH, D), jnp.float32)] * 3
            + [pltpu.VMEM((CH,), jnp.float32),
               pltpu.VMEM_SHARED((NPAD, D), jnp.float32)]
            + [pltpu.SemaphoreType.DMA] * 6),
    )(srcp, dstp, eap, q)


_BR = 1000


def _ln(v, g, b):
    mu = jnp.mean(v, axis=-1, keepdims=True)
    var = jnp.mean((v - mu) * (v - mu), axis=-1, keepdims=True)
    return (v - mu) * lax.rsqrt(var + 1e-5) * g + b


def _tc_prep(h_ref, g_ref, b_ref, t_ref, r_ref, p_ref):
    h = h_ref[...]
    r = jnp.maximum(_ln(h, g_ref[...], b_ref[...]), 0.0)
    m = r + EPS
    ex = jnp.exp(t_ref[...] * m)
    r_ref[...] = r
    p_ref[0, :, :] = ex
    p_ref[1, :, :] = ex * m


def _prep_call(h, g, b, trow):
    row = lambda i: (i, 0)
    one = lambda i: (0, 0)
    return pl.pallas_call(
        _tc_prep,
        grid=(N // _BR,),
        in_specs=[
            pl.BlockSpec((_BR, D), row),
            pl.BlockSpec((1, D), one),
            pl.BlockSpec((1, D), one),
            pl.BlockSpec((1, D), one),
        ],
        out_specs=[
            pl.BlockSpec((_BR, D), row),
            pl.BlockSpec((2, _BR, D), lambda i: (0, i, 0)),
        ],
        out_shape=[
            jax.ShapeDtypeStruct((N, D), jnp.float32),
            jax.ShapeDtypeStruct((2, N, D), jnp.float32),
        ],
    )(h, g, b, trow)


def _tc_mlp(h_ref, r_ref, den_ref, num_ref, w1_ref, b1_ref, lg_ref, lb_ref,
            w2_ref, b2_ref, h2_ref):
    aggr = num_ref[...] / (den_ref[...] + 1e-16)
    out = aggr + r_ref[...]
    z = jnp.dot(out, w1_ref[...], preferred_element_type=jnp.float32)
    z = _ln(z + b1_ref[...], lg_ref[...], lb_ref[...])
    z = jnp.maximum(z, 0.0)
    z2 = jnp.dot(z, w2_ref[...], preferred_element_type=jnp.float32)
    h2_ref[...] = h_ref[...] + z2 + b2_ref[...]


def _mlp_call(h, r, den, num, w1, b1, lg, lb, w2, b2):
    row = lambda i: (i, 0)
    one = lambda i: (0, 0)
    return pl.pallas_call(
        _tc_mlp,
        grid=(N // _BR,),
        in_specs=[
            pl.BlockSpec((_BR, D), row),
            pl.BlockSpec((_BR, D), row),
            pl.BlockSpec((_BR, D), row),
            pl.BlockSpec((_BR, D), row),
            pl.BlockSpec((D, H), one),
            pl.BlockSpec((1, H), one),
            pl.BlockSpec((1, H), one),
            pl.BlockSpec((1, H), one),
            pl.BlockSpec((H, D), one),
            pl.BlockSpec((1, D), one),
        ],
        out_specs=pl.BlockSpec((_BR, D), row),
        out_shape=jax.ShapeDtypeStruct((N, D), jnp.float32),
    )(h, r, den, num, w1, b1, lg, lb, w2, b2)


def _tc_degsum(d_ref, o_ref):
    d = d_ref[...]
    o_ref[...] = d[0:NPAD, 0:1] + d[NPAD:2 * NPAD, 0:1] + 1.0


def _degsum_call(deg2):
    return pl.pallas_call(
        _tc_degsum,
        out_shape=jax.ShapeDtypeStruct((NPAD, 1), jnp.float32),
    )(deg2)


def _tc_gcnprep(h_ref, wg_ref, deg_ref, q_ref, dis_ref, st_ref):
    hw = jnp.dot(h_ref[...], wg_ref[...], preferred_element_type=jnp.float32)
    deg = deg_ref[...]
    dis = jnp.where(deg > 0, 1.0 / jnp.sqrt(deg), 0.0)
    q_ref[...] = hw * dis
    dis_ref[...] = dis
    st_ref[...] = hw * (dis * dis)


def _gcnprep_call(h, wgp, degcol):
    row = lambda i: (i, 0)
    one = lambda i: (0, 0)
    return pl.pallas_call(
        _tc_gcnprep,
        grid=(N // _BR,),
        in_specs=[
            pl.BlockSpec((_BR, D), row),
            pl.BlockSpec((D, D), one),
            pl.BlockSpec((_BR, 1), row),
        ],
        out_specs=[
            pl.BlockSpec((_BR, D), row),
            pl.BlockSpec((_BR, 1), row),
            pl.BlockSpec((_BR, D), row),
        ],
        out_shape=[
            jax.ShapeDtypeStruct((N, D), jnp.float32),
            jax.ShapeDtypeStruct((N, 1), jnp.float32),
            jax.ShapeDtypeStruct((N, D), jnp.float32),
        ],
    )(h, wgp, degcol)


def _tc_final(e0_ref, e1_ref, dis_ref, st_ref, bg_ref, o_ref):
    o = dis_ref[...] * (e0_ref[...] + e1_ref[...]) + st_ref[...] + bg_ref[...]
    mx = jnp.max(o, axis=-1, keepdims=True)
    lse = jnp.log(jnp.sum(jnp.exp(o - mx), axis=-1, keepdims=True))
    o_ref[...] = o - mx - lse


def _final_call(e0, e1, dis, st, bgp):
    row = lambda i: (i, 0)
    one = lambda i: (0, 0)
    return pl.pallas_call(
        _tc_final,
        grid=(N // _BR,),
        in_specs=[
            pl.BlockSpec((_BR, D), row),
            pl.BlockSpec((_BR, D), row),
            pl.BlockSpec((_BR, 1), row),
            pl.BlockSpec((_BR, D), row),
            pl.BlockSpec((1, D), one),
        ],
        out_specs=pl.BlockSpec((_BR, D), row),
        out_shape=jax.ShapeDtypeStruct((N, D), jnp.float32),
    )(e0, e1, dis, st, bgp)


def kernel(x, edge_index, edge_attr, emb, ln_g, ln_b, t, W1, b1,
           mlp_ln_g, mlp_ln_b, W2, b2, Wg, bg):
    f32 = jnp.float32
    src = edge_index[0]
    dst = edge_index[1]
    e = src.shape[0]
    npad_e = E_PAD - e
    srcp = jnp.concatenate([src, jnp.zeros((npad_e,), src.dtype)])
    dstp = jnp.concatenate([dst, jnp.full((npad_e,), N, dst.dtype)])
    eap = jnp.concatenate([edge_attr, jnp.zeros((npad_e,), f32)])
    idxp = jnp.concatenate(
        [x[:, 0].astype(jnp.int32),
         jnp.zeros((IDX_PAD - x.shape[0],), jnp.int32)])

    hfull = _emb_call(idxp, emb)
    h = hfull[:N]
    deg2 = _deg_call(dstp, eap)
    degcol = _degsum_call(deg2)[:N]

    for i in range(2):
        trow = jnp.full((1, D), t[i], f32)
        r, p2 = _prep_call(h, ln_g[i].reshape(1, D), ln_b[i].reshape(1, D),
                           trow)
        s2 = _edge_call(srcp, dstp, p2.reshape(2 * N, D))
        den = s2[:N]
        num = s2[NPAD:NPAD + N]
        h = _mlp_call(h, r, den, num, W1[i], b1[i].reshape(1, H),
                      mlp_ln_g[i].reshape(1, H), mlp_ln_b[i].reshape(1, H),
                      W2[i], b2[i].reshape(1, D))

    wgp = jnp.concatenate([Wg, jnp.zeros((D, D - NC), f32)], axis=1)
    q, dis, st = _gcnprep_call(h, wgp, degcol)
    eacc = _gcn_call(srcp, dstp, eap, q)
    e0 = eacc[:N]
    e1 = eacc[NPAD:NPAD + N]
    bgp = jnp.concatenate([bg, jnp.full((D - NC,), -1e30, f32)])
    out = _final_call(e0, e1, dis, st, bgp.reshape(1, D))
    return out[:, :NC]
